# Initial kernel scaffold; baseline (speedup 1.0000x reference)
import functools

import jax
import jax.numpy as jnp
from jax import lax
from jax.experimental import pallas as pl
from jax.experimental.pallas import tpu as pltpu
from jax.experimental.pallas import tpu_sc as plsc

_N = 10000
_E = 160000
_LAT = 128
_NC, _NS = 2, 16
_NW = _NC * _NS
_ROWS_W = _E // _NW
_CHUNK = 200
_NCHUNK = _ROWS_W // _CHUNK
_NODE_W = _N // _NS
_BN = 1000
_BE = 2000
_F32 = jnp.float32


def _sc_mesh():
    return plsc.VectorSubcoreMesh(core_axis_name="c", subcore_axis_name="s",
                                  num_cores=_NC, num_subcores=_NS)


def _sc_gather(nlS, nlR, snd, rcv, lastpos=None):
    with_pos = lastpos is not None
    out_type = [jax.ShapeDtypeStruct((_E, _LAT), _F32),
                jax.ShapeDtypeStruct((_E, _LAT), _F32)]
    scratch = [pltpu.VMEM((_CHUNK,), jnp.int32),
               pltpu.VMEM((_CHUNK,), jnp.int32),
               pltpu.VMEM((_CHUNK, _LAT), _F32),
               pltpu.VMEM((_CHUNK, _LAT), _F32)]
    if with_pos:
        out_type += [jax.ShapeDtypeStruct((_E, 16), _F32),
                     jax.ShapeDtypeStruct((_E, 16), _F32)]
        scratch += [pltpu.VMEM((_CHUNK, 16), _F32),
                    pltpu.VMEM((_CHUNK, 16), _F32)]

    @functools.partial(pl.kernel, out_type=out_type, mesh=_sc_mesh(),
                       scratch_types=scratch)
    def k(*refs):
        if with_pos:
            (nlS_h, nlR_h, snd_h, rcv_h, lp_h, gS_h, gR_h, gLS_h, gLR_h,
             idx_s, idx_r, bufS, bufR, bufLS, bufLR) = refs
        else:
            (nlS_h, nlR_h, snd_h, rcv_h, gS_h, gR_h,
             idx_s, idx_r, bufS, bufR) = refs
        wid = lax.axis_index("s") * _NC + lax.axis_index("c")
        base0 = wid * _ROWS_W

        @pl.loop(0, _NCHUNK)
        def _(c):
            base = base0 + c * _CHUNK
            pltpu.sync_copy(snd_h.at[pl.ds(base, _CHUNK)], idx_s)
            pltpu.sync_copy(rcv_h.at[pl.ds(base, _CHUNK)], idx_r)
            pltpu.sync_copy(nlS_h.at[idx_s], bufS)
            pltpu.sync_copy(nlR_h.at[idx_r], bufR)
            pltpu.sync_copy(bufS, gS_h.at[pl.ds(base, _CHUNK)])
            pltpu.sync_copy(bufR, gR_h.at[pl.ds(base, _CHUNK)])
            if with_pos:
                pltpu.sync_copy(lp_h.at[idx_s], bufLS)
                pltpu.sync_copy(lp_h.at[idx_r], bufLR)
                pltpu.sync_copy(bufLS, gLS_h.at[pl.ds(base, _CHUNK)])
                pltpu.sync_copy(bufLR, gLR_h.at[pl.ds(base, _CHUNK)])

    if with_pos:
        return k(nlS, nlR, snd, rcv, lastpos)
    return k(nlS, nlR, snd, rcv)


def _sc_segment_sum(e_lat, rcv, zeros):
    @functools.partial(
        pl.kernel,
        out_type=jax.ShapeDtypeStruct((_NC, _N, _LAT), _F32),
        mesh=_sc_mesh(),
        scratch_types=[pltpu.VMEM_SHARED((_N, _LAT), _F32),
                       pltpu.VMEM((_CHUNK, _LAT), _F32),
                       pltpu.VMEM((_CHUNK,), jnp.int32)])
    def k(e_h, rcv_h, z_h, out_h, acc_sh, buf, idx):
        cid = lax.axis_index("c")
        sid = lax.axis_index("s")
        pltpu.sync_copy(z_h.at[pl.ds(sid * _NODE_W, _NODE_W)],
                        acc_sh.at[pl.ds(sid * _NODE_W, _NODE_W)])
        plsc.subcore_barrier()
        base0 = cid * (_E // _NC) + sid * (_E // _NC // _NS)

        @pl.loop(0, _NCHUNK)
        def _(c):
            base = base0 + c * _CHUNK
            pltpu.sync_copy(rcv_h.at[pl.ds(base, _CHUNK)], idx)
            pltpu.sync_copy(e_h.at[pl.ds(base, _CHUNK)], buf)
            pltpu.sync_copy(buf, acc_sh.at[idx], add=True)

        plsc.subcore_barrier()
        pltpu.sync_copy(acc_sh.at[pl.ds(sid * _NODE_W, _NODE_W)],
                        out_h.at[cid, pl.ds(sid * _NODE_W, _NODE_W)])

    return k(e_lat, rcv, zeros)


def _dot(a, b):
    return jnp.dot(a, b, preferred_element_type=_F32)


def _w_spec():
    return pl.BlockSpec((_LAT, _LAT), lambda i: (0, 0))


def _b_spec():
    return pl.BlockSpec((_LAT,), lambda i: (0,))


def _tc_node_encode(pos_flat, types_b, emb_table, Wv, We, b_enc, Wm2, Wm3):
    def body(pos_ref, t_ref, emb_ref, Wv_ref, We_ref, b_ref, W2_ref, W3_ref,
             nl_ref, s_ref, r_ref):
        pos = pos_ref[...]
        vel = pos[:, 2:12] - pos[:, 0:10]
        emb_proj = _dot(emb_ref[...], We_ref[...])
        t = t_ref[...]
        pre = _dot(vel, Wv_ref[...]) + b_ref[...][None, :]
        for kk in range(9):
            pre = pre + jnp.where(t == kk, emb_proj[kk][None, :], 0.0)
        nl = jnp.maximum(pre, 0.0)
        nl_ref[...] = nl
        s_ref[...] = _dot(nl, W2_ref[...])
        r_ref[...] = _dot(nl, W3_ref[...])

    grid = (_N // _BN,)
    return pl.pallas_call(
        body,
        grid=grid,
        in_specs=[
            pl.BlockSpec((_BN, 12), lambda i: (i, 0)),
            pl.BlockSpec((_BN, _LAT), lambda i: (i, 0)),
            pl.BlockSpec((9, 16), lambda i: (0, 0)),
            pl.BlockSpec((10, _LAT), lambda i: (0, 0)),
            pl.BlockSpec((16, _LAT), lambda i: (0, 0)),
            _b_spec(),
            _w_spec(),
            _w_spec(),
        ],
        out_specs=[pl.BlockSpec((_BN, _LAT), lambda i: (i, 0))] * 3,
        out_shape=[jax.ShapeDtypeStruct((_N, _LAT), _F32)] * 3,
    )(pos_flat, types_b, emb_table, Wv, We, b_enc, Wm2, Wm3)


def _tc_edge_step1(gS, gR, gLS, gLR, W_edge_enc, b_edge_enc, Wm1, b_msg):
    def body(gS_ref, gR_ref, gLS_ref, gLR_ref, We_ref, be_ref, W1_ref, bm_ref,
             out_ref):
        rel = gLS_ref[...] - gLR_ref[...]
        x = rel[:, 0:1]
        y = rel[:, 1:2]
        d = jnp.sqrt(x * x + y * y)
        We = We_ref[...]
        e0 = jnp.maximum(x * We[0][None, :] + y * We[1][None, :]
                         + d * We[2][None, :] + be_ref[...][None, :], 0.0)
        pre = _dot(e0, W1_ref[...]) + bm_ref[...][None, :] \
            + gS_ref[...] + gR_ref[...]
        out_ref[...] = e0 + jnp.maximum(pre, 0.0)

    grid = (_E // _BE,)
    return pl.pallas_call(
        body,
        grid=grid,
        in_specs=[
            pl.BlockSpec((_BE, _LAT), lambda i: (i, 0)),
            pl.BlockSpec((_BE, _LAT), lambda i: (i, 0)),
            pl.BlockSpec((_BE, 16), lambda i: (i, 0)),
            pl.BlockSpec((_BE, 16), lambda i: (i, 0)),
            pl.BlockSpec((3, _LAT), lambda i: (0, 0)),
            _b_spec(),
            _w_spec(),
            _b_spec(),
        ],
        out_specs=pl.BlockSpec((_BE, _LAT), lambda i: (i, 0)),
        out_shape=jax.ShapeDtypeStruct((_E, _LAT), _F32),
    )(gS, gR, gLS, gLR, W_edge_enc, b_edge_enc, Wm1, b_msg)


def _tc_edge_step2(e_prev, gS, gR, Wm1, b_msg):
    def body(e_ref, gS_ref, gR_ref, W1_ref, bm_ref, out_ref):
        e = e_ref[...]
        pre = _dot(e, W1_ref[...]) + bm_ref[...][None, :] \
            + gS_ref[...] + gR_ref[...]
        out_ref[...] = e + jnp.maximum(pre, 0.0)

    grid = (_E // _BE,)
    return pl.pallas_call(
        body,
        grid=grid,
        in_specs=[
            pl.BlockSpec((_BE, _LAT), lambda i: (i, 0)),
            pl.BlockSpec((_BE, _LAT), lambda i: (i, 0)),
            pl.BlockSpec((_BE, _LAT), lambda i: (i, 0)),
            _w_spec(),
            _b_spec(),
        ],
        out_specs=pl.BlockSpec((_BE, _LAT), lambda i: (i, 0)),
        out_shape=jax.ShapeDtypeStruct((_E, _LAT), _F32),
    )(e_prev, gS, gR, Wm1, b_msg)


def _tc_node_update(node_lat, partials, Wu1, Wu2, b_upd, Wm2, Wm3):
    def body(nl_ref, p_ref, Wu1_ref, Wu2_ref, bu_ref, W2_ref, W3_ref,
             nl2_ref, s_ref, r_ref):
        nl = nl_ref[...]
        agg = p_ref[0] + p_ref[1]
        nl2 = nl + jnp.maximum(
            _dot(nl, Wu1_ref[...]) + _dot(agg, Wu2_ref[...])
            + bu_ref[...][None, :], 0.0)
        nl2_ref[...] = nl2
        s_ref[...] = _dot(nl2, W2_ref[...])
        r_ref[...] = _dot(nl2, W3_ref[...])

    grid = (_N // _BN,)
    return pl.pallas_call(
        body,
        grid=grid,
        in_specs=[
            pl.BlockSpec((_BN, _LAT), lambda i: (i, 0)),
            pl.BlockSpec((_NC, _BN, _LAT), lambda i: (0, i, 0)),
            _w_spec(), _w_spec(), _b_spec(), _w_spec(), _w_spec(),
        ],
        out_specs=[pl.BlockSpec((_BN, _LAT), lambda i: (i, 0))] * 3,
        out_shape=[jax.ShapeDtypeStruct((_N, _LAT), _F32)] * 3,
    )(node_lat, partials, Wu1, Wu2, b_upd, Wm2, Wm3)


def _tc_node_final(node_lat, partials, Wu1, Wu2, b_upd, Wd_pad, bd_pad):
    def body(nl_ref, p_ref, Wu1_ref, Wu2_ref, bu_ref, Wd_ref, bd_ref,
             out_ref):
        nl = nl_ref[...]
        agg = p_ref[0] + p_ref[1]
        nl2 = nl + jnp.maximum(
            _dot(nl, Wu1_ref[...]) + _dot(agg, Wu2_ref[...])
            + bu_ref[...][None, :], 0.0)
        out_ref[...] = _dot(nl2, Wd_ref[...]) + bd_ref[...][None, :]

    grid = (_N // _BN,)
    return pl.pallas_call(
        body,
        grid=grid,
        in_specs=[
            pl.BlockSpec((_BN, _LAT), lambda i: (i, 0)),
            pl.BlockSpec((_NC, _BN, _LAT), lambda i: (0, i, 0)),
            _w_spec(), _w_spec(), _b_spec(), _w_spec(), _b_spec(),
        ],
        out_specs=pl.BlockSpec((_BN, _LAT), lambda i: (i, 0)),
        out_shape=jax.ShapeDtypeStruct((_N, _LAT), _F32),
    )(node_lat, partials, Wu1, Wu2, b_upd, Wd_pad, bd_pad)


def kernel(position_sequence, particle_types, edge_index, emb_table,
           W_node_enc, b_node_enc, W_edge_enc, b_edge_enc,
           W_msg, b_msg, W_upd, b_upd, W_dec, b_dec):
    pos_flat = position_sequence.reshape(_N, 12)
    types_b = jnp.broadcast_to(
        particle_types.astype(jnp.int32)[:, None], (_N, _LAT))
    snd = edge_index[0].astype(jnp.int32)
    rcv = edge_index[1].astype(jnp.int32)
    lastpos = jnp.pad(pos_flat[:, 10:12], ((0, 0), (0, 14)))
    Wv = W_node_enc[:10]
    We = W_node_enc[10:]
    Wm1 = W_msg[:_LAT]
    Wm2 = W_msg[_LAT:2 * _LAT]
    Wm3 = W_msg[2 * _LAT:]
    Wu1 = W_upd[:_LAT]
    Wu2 = W_upd[_LAT:]
    Wd_pad = jnp.pad(W_dec, ((0, 0), (0, _LAT - W_dec.shape[1])))
    bd_pad = jnp.pad(b_dec, (0, _LAT - b_dec.shape[0]))
    zeros = jnp.zeros((_N, _LAT), _F32)

    nl0, nlS0, nlR0 = _tc_node_encode(
        pos_flat, types_b, emb_table, Wv, We, b_node_enc, Wm2, Wm3)

    gS1, gR1, gLS, gLR = _sc_gather(nlS0, nlR0, snd, rcv, lastpos)
    e1 = _tc_edge_step1(gS1, gR1, gLS, gLR, W_edge_enc, b_edge_enc,
                        Wm1, b_msg)
    p1 = _sc_segment_sum(e1, rcv, zeros)
    nl1, nlS1, nlR1 = _tc_node_update(nl0, p1, Wu1, Wu2, b_upd, Wm2, Wm3)

    gS2, gR2 = _sc_gather(nlS1, nlR1, snd, rcv)
    e2 = _tc_edge_step2(e1, gS2, gR2, Wm1, b_msg)
    p2 = _sc_segment_sum(e2, rcv, zeros)
    out_pad = _tc_node_final(nl1, p2, Wu1, Wu2, b_upd, Wd_pad, bd_pad)

    return out_pad[:, :W_dec.shape[1]]


# speedup vs baseline: 3.5741x; 3.5741x over previous
import functools

import jax
import jax.numpy as jnp
from jax import lax
from jax.experimental import pallas as pl
from jax.experimental.pallas import tpu as pltpu
from jax.experimental.pallas import tpu_sc as plsc

_N = 10000
_E = 160000
_LAT = 128
_NC, _NS = 2, 16
_NW = _NC * _NS
_ROWS_W = _E // _NW
_CHUNK = 200
_NCHUNK = _ROWS_W // _CHUNK
_NODE_W = _N // _NS
_BN = 1000
_BE = 2000
_F32 = jnp.float32


def _sc_mesh():
    return plsc.VectorSubcoreMesh(core_axis_name="c", subcore_axis_name="s",
                                  num_cores=_NC, num_subcores=_NS)


def _sc_gather(nlS, nlR, snd, rcv, lastpos=None):
    with_pos = lastpos is not None
    out_type = [jax.ShapeDtypeStruct((_E, _LAT), _F32),
                jax.ShapeDtypeStruct((_E, _LAT), _F32)]
    scratch = [pltpu.VMEM((_CHUNK,), jnp.int32),
               pltpu.VMEM((_CHUNK,), jnp.int32),
               pltpu.VMEM((_CHUNK, _LAT), _F32),
               pltpu.VMEM((_CHUNK, _LAT), _F32)]
    if with_pos:
        out_type += [jax.ShapeDtypeStruct((_E, 16), _F32),
                     jax.ShapeDtypeStruct((_E, 16), _F32)]
        scratch += [pltpu.VMEM((_CHUNK, 16), _F32),
                    pltpu.VMEM((_CHUNK, 16), _F32)]

    @functools.partial(pl.kernel, out_type=out_type, mesh=_sc_mesh(),
                       scratch_types=scratch,
                       compiler_params=pltpu.CompilerParams(
                           use_tc_tiling_on_sc=False))
    def k(*refs):
        if with_pos:
            (nlS_h, nlR_h, snd_h, rcv_h, lp_h, gS_h, gR_h, gLS_h, gLR_h,
             idx_s, idx_r, bufS, bufR, bufLS, bufLR) = refs
        else:
            (nlS_h, nlR_h, snd_h, rcv_h, gS_h, gR_h,
             idx_s, idx_r, bufS, bufR) = refs
        wid = lax.axis_index("s") * _NC + lax.axis_index("c")
        base0 = wid * _ROWS_W

        @pl.loop(0, _NCHUNK)
        def _(c):
            base = base0 + c * _CHUNK
            pltpu.sync_copy(snd_h.at[pl.ds(base, _CHUNK)], idx_s)
            pltpu.sync_copy(rcv_h.at[pl.ds(base, _CHUNK)], idx_r)
            pltpu.sync_copy(nlS_h.at[idx_s], bufS)
            pltpu.sync_copy(nlR_h.at[idx_r], bufR)
            pltpu.sync_copy(bufS, gS_h.at[pl.ds(base, _CHUNK)])
            pltpu.sync_copy(bufR, gR_h.at[pl.ds(base, _CHUNK)])
            if with_pos:
                pltpu.sync_copy(lp_h.at[idx_s], bufLS)
                pltpu.sync_copy(lp_h.at[idx_r], bufLR)
                pltpu.sync_copy(bufLS, gLS_h.at[pl.ds(base, _CHUNK)])
                pltpu.sync_copy(bufLR, gLR_h.at[pl.ds(base, _CHUNK)])

    if with_pos:
        return k(nlS, nlR, snd, rcv, lastpos)
    return k(nlS, nlR, snd, rcv)


def _sc_segment_sum(e_lat, rcv, zeros):
    @functools.partial(
        pl.kernel,
        out_type=jax.ShapeDtypeStruct((_NC, _N, _LAT), _F32),
        mesh=_sc_mesh(),
        scratch_types=[pltpu.VMEM_SHARED((_N, _LAT), _F32),
                       pltpu.VMEM((_CHUNK, _LAT), _F32),
                       pltpu.VMEM((_CHUNK,), jnp.int32)])
    def k(e_h, rcv_h, z_h, out_h, acc_sh, buf, idx):
        cid = lax.axis_index("c")
        sid = lax.axis_index("s")
        pltpu.sync_copy(z_h.at[pl.ds(sid * 624, 624)],
                        acc_sh.at[pl.ds(sid * 624, 624)])

        @pl.when(sid == _NS - 1)
        def _():
            pltpu.sync_copy(z_h.at[pl.ds(_NS * 624, _N - _NS * 624)],
                            acc_sh.at[pl.ds(_NS * 624, _N - _NS * 624)])

        plsc.subcore_barrier()
        base0 = cid * (_E // _NC) + sid * (_E // _NC // _NS)

        @pl.loop(0, _NCHUNK)
        def _(c):
            base = base0 + c * _CHUNK
            pltpu.sync_copy(rcv_h.at[pl.ds(base, _CHUNK)], idx)
            pltpu.sync_copy(e_h.at[pl.ds(base, _CHUNK)], buf)
            pltpu.sync_copy(buf, acc_sh.at[idx], add=True)

        plsc.subcore_barrier()
        pltpu.sync_copy(acc_sh.at[pl.ds(sid * 624, 624)],
                        out_h.at[cid, pl.ds(sid * 624, 624)])

        @pl.when(sid == _NS - 1)
        def _():
            pltpu.sync_copy(acc_sh.at[pl.ds(_NS * 624, _N - _NS * 624)],
                            out_h.at[cid, pl.ds(_NS * 624, _N - _NS * 624)])

    return k(e_lat, rcv, zeros)


def _dot(a, b):
    return jnp.dot(a, b, preferred_element_type=_F32)


def _w_spec():
    return pl.BlockSpec((_LAT, _LAT), lambda i: (0, 0))


def _b_spec():
    return pl.BlockSpec((_LAT,), lambda i: (0,))


def _tc_node_encode(pos_flat, types_b, emb_table, Wv, We, b_enc, Wm2, Wm3):
    def body(pos_ref, t_ref, emb_ref, Wv_ref, We_ref, b_ref, W2_ref, W3_ref,
             nl_ref, s_ref, r_ref):
        pos = pos_ref[...]
        vel = pos[:, 2:12] - pos[:, 0:10]
        emb_proj = _dot(emb_ref[...], We_ref[...])
        t = t_ref[...]
        pre = _dot(vel, Wv_ref[...]) + b_ref[...][None, :]
        for kk in range(9):
            pre = pre + jnp.where(t == kk, emb_proj[kk][None, :], 0.0)
        nl = jnp.maximum(pre, 0.0)
        nl_ref[...] = nl
        s_ref[...] = _dot(nl, W2_ref[...])
        r_ref[...] = _dot(nl, W3_ref[...])

    grid = (_N // _BN,)
    return pl.pallas_call(
        body,
        grid=grid,
        in_specs=[
            pl.BlockSpec((_BN, 12), lambda i: (i, 0)),
            pl.BlockSpec((_BN, _LAT), lambda i: (i, 0)),
            pl.BlockSpec((9, 16), lambda i: (0, 0)),
            pl.BlockSpec((10, _LAT), lambda i: (0, 0)),
            pl.BlockSpec((16, _LAT), lambda i: (0, 0)),
            _b_spec(),
            _w_spec(),
            _w_spec(),
        ],
        out_specs=[pl.BlockSpec((_BN, _LAT), lambda i: (i, 0))] * 3,
        out_shape=[jax.ShapeDtypeStruct((_N, _LAT), _F32)] * 3,
    )(pos_flat, types_b, emb_table, Wv, We, b_enc, Wm2, Wm3)


def _tc_edge_step1(gS, gR, gLS, gLR, W_edge_enc, b_edge_enc, Wm1, b_msg):
    def body(gS_ref, gR_ref, gLS_ref, gLR_ref, We_ref, be_ref, W1_ref, bm_ref,
             out_ref):
        rel = gLS_ref[...] - gLR_ref[...]
        x = rel[:, 0:1]
        y = rel[:, 1:2]
        d = jnp.sqrt(x * x + y * y)
        We = We_ref[...]
        e0 = jnp.maximum(x * We[0][None, :] + y * We[1][None, :]
                         + d * We[2][None, :] + be_ref[...][None, :], 0.0)
        pre = _dot(e0, W1_ref[...]) + bm_ref[...][None, :] \
            + gS_ref[...] + gR_ref[...]
        out_ref[...] = e0 + jnp.maximum(pre, 0.0)

    grid = (_E // _BE,)
    return pl.pallas_call(
        body,
        grid=grid,
        in_specs=[
            pl.BlockSpec((_BE, _LAT), lambda i: (i, 0)),
            pl.BlockSpec((_BE, _LAT), lambda i: (i, 0)),
            pl.BlockSpec((_BE, 16), lambda i: (i, 0)),
            pl.BlockSpec((_BE, 16), lambda i: (i, 0)),
            pl.BlockSpec((3, _LAT), lambda i: (0, 0)),
            _b_spec(),
            _w_spec(),
            _b_spec(),
        ],
        out_specs=pl.BlockSpec((_BE, _LAT), lambda i: (i, 0)),
        out_shape=jax.ShapeDtypeStruct((_E, _LAT), _F32),
    )(gS, gR, gLS, gLR, W_edge_enc, b_edge_enc, Wm1, b_msg)


def _tc_edge_step2(e_prev, gS, gR, Wm1, b_msg):
    def body(e_ref, gS_ref, gR_ref, W1_ref, bm_ref, out_ref):
        e = e_ref[...]
        pre = _dot(e, W1_ref[...]) + bm_ref[...][None, :] \
            + gS_ref[...] + gR_ref[...]
        out_ref[...] = e + jnp.maximum(pre, 0.0)

    grid = (_E // _BE,)
    return pl.pallas_call(
        body,
        grid=grid,
        in_specs=[
            pl.BlockSpec((_BE, _LAT), lambda i: (i, 0)),
            pl.BlockSpec((_BE, _LAT), lambda i: (i, 0)),
            pl.BlockSpec((_BE, _LAT), lambda i: (i, 0)),
            _w_spec(),
            _b_spec(),
        ],
        out_specs=pl.BlockSpec((_BE, _LAT), lambda i: (i, 0)),
        out_shape=jax.ShapeDtypeStruct((_E, _LAT), _F32),
    )(e_prev, gS, gR, Wm1, b_msg)


def _tc_node_update(node_lat, partials, Wu1, Wu2, b_upd, Wm2, Wm3):
    def body(nl_ref, p_ref, Wu1_ref, Wu2_ref, bu_ref, W2_ref, W3_ref,
             nl2_ref, s_ref, r_ref):
        nl = nl_ref[...]
        agg = p_ref[0] + p_ref[1]
        nl2 = nl + jnp.maximum(
            _dot(nl, Wu1_ref[...]) + _dot(agg, Wu2_ref[...])
            + bu_ref[...][None, :], 0.0)
        nl2_ref[...] = nl2
        s_ref[...] = _dot(nl2, W2_ref[...])
        r_ref[...] = _dot(nl2, W3_ref[...])

    grid = (_N // _BN,)
    return pl.pallas_call(
        body,
        grid=grid,
        in_specs=[
            pl.BlockSpec((_BN, _LAT), lambda i: (i, 0)),
            pl.BlockSpec((_NC, _BN, _LAT), lambda i: (0, i, 0)),
            _w_spec(), _w_spec(), _b_spec(), _w_spec(), _w_spec(),
        ],
        out_specs=[pl.BlockSpec((_BN, _LAT), lambda i: (i, 0))] * 3,
        out_shape=[jax.ShapeDtypeStruct((_N, _LAT), _F32)] * 3,
    )(node_lat, partials, Wu1, Wu2, b_upd, Wm2, Wm3)


def _tc_node_final(node_lat, partials, Wu1, Wu2, b_upd, Wd_pad, bd_pad):
    def body(nl_ref, p_ref, Wu1_ref, Wu2_ref, bu_ref, Wd_ref, bd_ref,
             out_ref):
        nl = nl_ref[...]
        agg = p_ref[0] + p_ref[1]
        nl2 = nl + jnp.maximum(
            _dot(nl, Wu1_ref[...]) + _dot(agg, Wu2_ref[...])
            + bu_ref[...][None, :], 0.0)
        out_ref[...] = _dot(nl2, Wd_ref[...]) + bd_ref[...][None, :]

    grid = (_N // _BN,)
    return pl.pallas_call(
        body,
        grid=grid,
        in_specs=[
            pl.BlockSpec((_BN, _LAT), lambda i: (i, 0)),
            pl.BlockSpec((_NC, _BN, _LAT), lambda i: (0, i, 0)),
            _w_spec(), _w_spec(), _b_spec(), _w_spec(), _b_spec(),
        ],
        out_specs=pl.BlockSpec((_BN, _LAT), lambda i: (i, 0)),
        out_shape=jax.ShapeDtypeStruct((_N, _LAT), _F32),
    )(node_lat, partials, Wu1, Wu2, b_upd, Wd_pad, bd_pad)


def kernel(position_sequence, particle_types, edge_index, emb_table,
           W_node_enc, b_node_enc, W_edge_enc, b_edge_enc,
           W_msg, b_msg, W_upd, b_upd, W_dec, b_dec):
    pos_flat = position_sequence.reshape(_N, 12)
    types_b = jnp.broadcast_to(
        particle_types.astype(jnp.int32)[:, None], (_N, _LAT))
    snd = edge_index[0].astype(jnp.int32)
    rcv = edge_index[1].astype(jnp.int32)
    lastpos = jnp.pad(pos_flat[:, 10:12], ((0, 0), (0, 14)))
    Wv = W_node_enc[:10]
    We = W_node_enc[10:]
    Wm1 = W_msg[:_LAT]
    Wm2 = W_msg[_LAT:2 * _LAT]
    Wm3 = W_msg[2 * _LAT:]
    Wu1 = W_upd[:_LAT]
    Wu2 = W_upd[_LAT:]
    Wd_pad = jnp.pad(W_dec, ((0, 0), (0, _LAT - W_dec.shape[1])))
    bd_pad = jnp.pad(b_dec, (0, _LAT - b_dec.shape[0]))
    zeros = jnp.zeros((_N, _LAT), _F32)

    nl0, nlS0, nlR0 = _tc_node_encode(
        pos_flat, types_b, emb_table, Wv, We, b_node_enc, Wm2, Wm3)

    gS1, gR1, gLS, gLR = _sc_gather(nlS0, nlR0, snd, rcv, lastpos)
    e1 = _tc_edge_step1(gS1, gR1, gLS, gLR, W_edge_enc, b_edge_enc,
                        Wm1, b_msg)
    p1 = _sc_segment_sum(e1, rcv, zeros)
    nl1, nlS1, nlR1 = _tc_node_update(nl0, p1, Wu1, Wu2, b_upd, Wm2, Wm3)

    gS2, gR2 = _sc_gather(nlS1, nlR1, snd, rcv)
    e2 = _tc_edge_step2(e1, gS2, gR2, Wm1, b_msg)
    p2 = _sc_segment_sum(e2, rcv, zeros)
    out_pad = _tc_node_final(nl1, p2, Wu1, Wu2, b_upd, Wd_pad, bd_pad)

    return out_pad[:, :W_dec.shape[1]]


# speedup vs baseline: 4.5395x; 1.2701x over previous
import functools

import jax
import jax.numpy as jnp
from jax import lax
from jax.experimental import pallas as pl
from jax.experimental.pallas import tpu as pltpu
from jax.experimental.pallas import tpu_sc as plsc

_N = 10000
_E = 160000
_LAT = 128
_NC, _NS = 2, 16
_NW = _NC * _NS
_ROWS_W = _E // _NW
_CHUNK = 200
_NCHUNK = _ROWS_W // _CHUNK
_NODE_W = _N // _NS
_BN = 1000
_BE = 2000
_F32 = jnp.float32


def _sc_mesh():
    return plsc.VectorSubcoreMesh(core_axis_name="c", subcore_axis_name="s",
                                  num_cores=_NC, num_subcores=_NS)


def _sc_gather(nlS, nlR, snd, rcv, lastpos=None):
    with_pos = lastpos is not None
    out_type = [jax.ShapeDtypeStruct((_E, _LAT), _F32),
                jax.ShapeDtypeStruct((_E, _LAT), _F32)]
    scratch = [pltpu.VMEM((_ROWS_W,), jnp.int32),
               pltpu.VMEM((_ROWS_W,), jnp.int32)]
    nbuf = 2
    per_buf = [pltpu.VMEM((_CHUNK, _LAT), _F32),
               pltpu.VMEM((_CHUNK, _LAT), _F32)]
    if with_pos:
        out_type += [jax.ShapeDtypeStruct((_E, 16), _F32),
                     jax.ShapeDtypeStruct((_E, 16), _F32)]
        per_buf += [pltpu.VMEM((_CHUNK, 16), _F32),
                    pltpu.VMEM((_CHUNK, 16), _F32)]
    nstream = len(per_buf)
    scratch += per_buf * nbuf
    scratch += [pltpu.SemaphoreType.DMA] * (2 * nbuf)

    @functools.partial(pl.kernel, out_type=out_type, mesh=_sc_mesh(),
                       scratch_types=scratch,
                       compiler_params=pltpu.CompilerParams(
                           use_tc_tiling_on_sc=False))
    def k(*refs):
        if with_pos:
            (nlS_h, nlR_h, snd_h, rcv_h, lp_h, gS_h, gR_h, gLS_h, gLR_h,
             idxS, idxR, *rest) = refs
            tabs = (nlS_h, nlR_h, lp_h, lp_h)
            outs = (gS_h, gR_h, gLS_h, gLR_h)
        else:
            (nlS_h, nlR_h, snd_h, rcv_h, gS_h, gR_h, idxS, idxR,
             *rest) = refs
            tabs = (nlS_h, nlR_h)
            outs = (gS_h, gR_h)
        bufs = [rest[b * nstream:(b + 1) * nstream] for b in range(nbuf)]
        sems = rest[nbuf * nstream:]
        sem_g = sems[:nbuf]
        sem_w = sems[nbuf:]
        idxs = (idxS, idxR, idxS, idxR)

        wid = lax.axis_index("s") * _NC + lax.axis_index("c")
        base0 = wid * _ROWS_W
        pltpu.sync_copy(snd_h.at[pl.ds(base0, _ROWS_W)], idxS)
        pltpu.sync_copy(rcv_h.at[pl.ds(base0, _ROWS_W)], idxR)

        def g_descs(c, b):
            ds_i = pl.ds(c * _CHUNK, _CHUNK)
            return [pltpu.make_async_copy(tabs[j].at[idxs[j].at[ds_i]],
                                          bufs[b][j], sem_g[b])
                    for j in range(nstream)]

        def w_descs(c, b):
            out_sl = pl.ds(base0 + c * _CHUNK, _CHUNK)
            return [pltpu.make_async_copy(bufs[b][j], outs[j].at[out_sl],
                                          sem_w[b])
                    for j in range(nstream)]

        def fire_g(c, b):
            for d in g_descs(c, b):
                d.start()

        fire_g(0, 0)
        fire_g(1, 1)

        def step(c, b):
            for d in g_descs(c, b):
                d.wait()
            for d in w_descs(c, b):
                d.start()
            for d in w_descs(c, b):
                d.wait()

            @pl.when(c + nbuf < _NCHUNK)
            def _():
                fire_g(c + nbuf, b)

        @pl.loop(0, _NCHUNK // nbuf)
        def _(cc):
            step(cc * nbuf, 0)
            step(cc * nbuf + 1, 1)

        if _NCHUNK % nbuf:
            step(_NCHUNK - 1, (_NCHUNK - 1) % nbuf)

    if with_pos:
        return k(nlS, nlR, snd, rcv, lastpos)
    return k(nlS, nlR, snd, rcv)


def _sc_segment_sum(e_lat, rcv, zeros):
    nring = 4
    chunk = 40
    nchunk = _ROWS_W // chunk

    @functools.partial(
        pl.kernel,
        out_type=jax.ShapeDtypeStruct((_NC, _N, _LAT), _F32),
        mesh=_sc_mesh(),
        scratch_types=[pltpu.VMEM_SHARED((_N, _LAT), _F32)]
        + [pltpu.VMEM((chunk,), jnp.int32)] * nring
        + [pltpu.VMEM((chunk, _LAT), _F32)] * nring
        + [pltpu.SemaphoreType.DMA] * (2 * nring))
    def k(e_h, rcv_h, z_h, out_h, acc_sh, *rest):
        idxs = rest[:nring]
        rows = rest[nring:2 * nring]
        semL = rest[2 * nring:3 * nring]
        semA = rest[3 * nring:]
        cid = lax.axis_index("c")
        sid = lax.axis_index("s")
        pltpu.sync_copy(z_h.at[pl.ds(sid * 624, 624)],
                        acc_sh.at[pl.ds(sid * 624, 624)])

        @pl.when(sid == _NS - 1)
        def _():
            pltpu.sync_copy(z_h.at[pl.ds(_NS * 624, _N - _NS * 624)],
                            acc_sh.at[pl.ds(_NS * 624, _N - _NS * 624)])

        plsc.subcore_barrier()
        base0 = cid * (_E // _NC) + sid * (_E // _NC // _NS)

        def load_descs(c, b):
            base = base0 + c * chunk
            return [pltpu.make_async_copy(rcv_h.at[pl.ds(base, chunk)],
                                          idxs[b], semL[b]),
                    pltpu.make_async_copy(e_h.at[pl.ds(base, chunk)],
                                          rows[b], semL[b])]

        def add_desc(b):
            return pltpu.make_async_copy(rows[b], acc_sh.at[idxs[b]],
                                         semA[b])

        def fire_loads(c, b):
            for d in load_descs(c, b):
                d.start()

        for b in range(nring):
            fire_loads(b, b)

        def step(c, b):
            for d in load_descs(c, b):
                d.wait()
            add_desc(b).start(add=True)
            add_desc(b).wait()

            @pl.when(c + nring < nchunk)
            def _():
                fire_loads(c + nring, b)

        @pl.loop(0, nchunk // nring)
        def _(cc):
            for b in range(nring):
                step(cc * nring + b, b)

        for c in range((nchunk // nring) * nring, nchunk):
            step(c, c % nring)

        plsc.subcore_barrier()
        pltpu.sync_copy(acc_sh.at[pl.ds(sid * 624, 624)],
                        out_h.at[cid, pl.ds(sid * 624, 624)])

        @pl.when(sid == _NS - 1)
        def _():
            pltpu.sync_copy(acc_sh.at[pl.ds(_NS * 624, _N - _NS * 624)],
                            out_h.at[cid, pl.ds(_NS * 624, _N - _NS * 624)])

    return k(e_lat, rcv, zeros)


def _dot(a, b):
    return jnp.dot(a, b, preferred_element_type=_F32)


def _w_spec():
    return pl.BlockSpec((_LAT, _LAT), lambda i: (0, 0))


def _b_spec():
    return pl.BlockSpec((_LAT,), lambda i: (0,))


def _tc_node_encode(pos_flat, types_b, emb_table, Wv, We, b_enc, Wm2, Wm3):
    def body(pos_ref, t_ref, emb_ref, Wv_ref, We_ref, b_ref, W2_ref, W3_ref,
             nl_ref, s_ref, r_ref):
        pos = pos_ref[...]
        vel = pos[:, 2:12] - pos[:, 0:10]
        emb_proj = _dot(emb_ref[...], We_ref[...])
        t = t_ref[...]
        pre = _dot(vel, Wv_ref[...]) + b_ref[...][None, :]
        for kk in range(9):
            pre = pre + jnp.where(t == kk, emb_proj[kk][None, :], 0.0)
        nl = jnp.maximum(pre, 0.0)
        nl_ref[...] = nl
        s_ref[...] = _dot(nl, W2_ref[...])
        r_ref[...] = _dot(nl, W3_ref[...])

    grid = (_N // _BN,)
    return pl.pallas_call(
        body,
        grid=grid,
        in_specs=[
            pl.BlockSpec((_BN, 12), lambda i: (i, 0)),
            pl.BlockSpec((_BN, _LAT), lambda i: (i, 0)),
            pl.BlockSpec((9, 16), lambda i: (0, 0)),
            pl.BlockSpec((10, _LAT), lambda i: (0, 0)),
            pl.BlockSpec((16, _LAT), lambda i: (0, 0)),
            _b_spec(),
            _w_spec(),
            _w_spec(),
        ],
        out_specs=[pl.BlockSpec((_BN, _LAT), lambda i: (i, 0))] * 3,
        out_shape=[jax.ShapeDtypeStruct((_N, _LAT), _F32)] * 3,
    )(pos_flat, types_b, emb_table, Wv, We, b_enc, Wm2, Wm3)


def _tc_edge_step1(gS, gR, gLS, gLR, W_edge_enc, b_edge_enc, Wm1, b_msg):
    def body(gS_ref, gR_ref, gLS_ref, gLR_ref, We_ref, be_ref, W1_ref, bm_ref,
             out_ref):
        rel = gLS_ref[...] - gLR_ref[...]
        x = rel[:, 0:1]
        y = rel[:, 1:2]
        d = jnp.sqrt(x * x + y * y)
        We = We_ref[...]
        e0 = jnp.maximum(x * We[0][None, :] + y * We[1][None, :]
                         + d * We[2][None, :] + be_ref[...][None, :], 0.0)
        pre = _dot(e0, W1_ref[...]) + bm_ref[...][None, :] \
            + gS_ref[...] + gR_ref[...]
        out_ref[...] = e0 + jnp.maximum(pre, 0.0)

    grid = (_E // _BE,)
    return pl.pallas_call(
        body,
        grid=grid,
        in_specs=[
            pl.BlockSpec((_BE, _LAT), lambda i: (i, 0)),
            pl.BlockSpec((_BE, _LAT), lambda i: (i, 0)),
            pl.BlockSpec((_BE, 16), lambda i: (i, 0)),
            pl.BlockSpec((_BE, 16), lambda i: (i, 0)),
            pl.BlockSpec((3, _LAT), lambda i: (0, 0)),
            _b_spec(),
            _w_spec(),
            _b_spec(),
        ],
        out_specs=pl.BlockSpec((_BE, _LAT), lambda i: (i, 0)),
        out_shape=jax.ShapeDtypeStruct((_E, _LAT), _F32),
    )(gS, gR, gLS, gLR, W_edge_enc, b_edge_enc, Wm1, b_msg)


def _tc_edge_step2(e_prev, gS, gR, Wm1, b_msg):
    def body(e_ref, gS_ref, gR_ref, W1_ref, bm_ref, out_ref):
        e = e_ref[...]
        pre = _dot(e, W1_ref[...]) + bm_ref[...][None, :] \
            + gS_ref[...] + gR_ref[...]
        out_ref[...] = e + jnp.maximum(pre, 0.0)

    grid = (_E // _BE,)
    return pl.pallas_call(
        body,
        grid=grid,
        in_specs=[
            pl.BlockSpec((_BE, _LAT), lambda i: (i, 0)),
            pl.BlockSpec((_BE, _LAT), lambda i: (i, 0)),
            pl.BlockSpec((_BE, _LAT), lambda i: (i, 0)),
            _w_spec(),
            _b_spec(),
        ],
        out_specs=pl.BlockSpec((_BE, _LAT), lambda i: (i, 0)),
        out_shape=jax.ShapeDtypeStruct((_E, _LAT), _F32),
    )(e_prev, gS, gR, Wm1, b_msg)


def _tc_node_update(node_lat, partials, Wu1, Wu2, b_upd, Wm2, Wm3):
    def body(nl_ref, p_ref, Wu1_ref, Wu2_ref, bu_ref, W2_ref, W3_ref,
             nl2_ref, s_ref, r_ref):
        nl = nl_ref[...]
        agg = p_ref[0] + p_ref[1]
        nl2 = nl + jnp.maximum(
            _dot(nl, Wu1_ref[...]) + _dot(agg, Wu2_ref[...])
            + bu_ref[...][None, :], 0.0)
        nl2_ref[...] = nl2
        s_ref[...] = _dot(nl2, W2_ref[...])
        r_ref[...] = _dot(nl2, W3_ref[...])

    grid = (_N // _BN,)
    return pl.pallas_call(
        body,
        grid=grid,
        in_specs=[
            pl.BlockSpec((_BN, _LAT), lambda i: (i, 0)),
            pl.BlockSpec((_NC, _BN, _LAT), lambda i: (0, i, 0)),
            _w_spec(), _w_spec(), _b_spec(), _w_spec(), _w_spec(),
        ],
        out_specs=[pl.BlockSpec((_BN, _LAT), lambda i: (i, 0))] * 3,
        out_shape=[jax.ShapeDtypeStruct((_N, _LAT), _F32)] * 3,
    )(node_lat, partials, Wu1, Wu2, b_upd, Wm2, Wm3)


def _tc_node_final(node_lat, partials, Wu1, Wu2, b_upd, Wd_pad, bd_pad):
    def body(nl_ref, p_ref, Wu1_ref, Wu2_ref, bu_ref, Wd_ref, bd_ref,
             out_ref):
        nl = nl_ref[...]
        agg = p_ref[0] + p_ref[1]
        nl2 = nl + jnp.maximum(
            _dot(nl, Wu1_ref[...]) + _dot(agg, Wu2_ref[...])
            + bu_ref[...][None, :], 0.0)
        out_ref[...] = _dot(nl2, Wd_ref[...]) + bd_ref[...][None, :]

    grid = (_N // _BN,)
    return pl.pallas_call(
        body,
        grid=grid,
        in_specs=[
            pl.BlockSpec((_BN, _LAT), lambda i: (i, 0)),
            pl.BlockSpec((_NC, _BN, _LAT), lambda i: (0, i, 0)),
            _w_spec(), _w_spec(), _b_spec(), _w_spec(), _b_spec(),
        ],
        out_specs=pl.BlockSpec((_BN, _LAT), lambda i: (i, 0)),
        out_shape=jax.ShapeDtypeStruct((_N, _LAT), _F32),
    )(node_lat, partials, Wu1, Wu2, b_upd, Wd_pad, bd_pad)


def kernel(position_sequence, particle_types, edge_index, emb_table,
           W_node_enc, b_node_enc, W_edge_enc, b_edge_enc,
           W_msg, b_msg, W_upd, b_upd, W_dec, b_dec):
    pos_flat = position_sequence.reshape(_N, 12)
    types_b = jnp.broadcast_to(
        particle_types.astype(jnp.int32)[:, None], (_N, _LAT))
    snd = edge_index[0].astype(jnp.int32)
    rcv = edge_index[1].astype(jnp.int32)
    lastpos = jnp.pad(pos_flat[:, 10:12], ((0, 0), (0, 14)))
    Wv = W_node_enc[:10]
    We = W_node_enc[10:]
    Wm1 = W_msg[:_LAT]
    Wm2 = W_msg[_LAT:2 * _LAT]
    Wm3 = W_msg[2 * _LAT:]
    Wu1 = W_upd[:_LAT]
    Wu2 = W_upd[_LAT:]
    Wd_pad = jnp.pad(W_dec, ((0, 0), (0, _LAT - W_dec.shape[1])))
    bd_pad = jnp.pad(b_dec, (0, _LAT - b_dec.shape[0]))
    zeros = jnp.zeros((_N, _LAT), _F32)

    nl0, nlS0, nlR0 = _tc_node_encode(
        pos_flat, types_b, emb_table, Wv, We, b_node_enc, Wm2, Wm3)

    gS1, gR1, gLS, gLR = _sc_gather(nlS0, nlR0, snd, rcv, lastpos)
    e1 = _tc_edge_step1(gS1, gR1, gLS, gLR, W_edge_enc, b_edge_enc,
                        Wm1, b_msg)
    p1 = _sc_segment_sum(e1, rcv, zeros)
    nl1, nlS1, nlR1 = _tc_node_update(nl0, p1, Wu1, Wu2, b_upd, Wm2, Wm3)

    gS2, gR2 = _sc_gather(nlS1, nlR1, snd, rcv)
    e2 = _tc_edge_step2(e1, gS2, gR2, Wm1, b_msg)
    p2 = _sc_segment_sum(e2, rcv, zeros)
    out_pad = _tc_node_final(nl1, p2, Wu1, Wu2, b_upd, Wd_pad, bd_pad)

    return out_pad[:, :W_dec.shape[1]]


# speedup vs baseline: 5.1252x; 1.1290x over previous
import functools

import jax
import jax.numpy as jnp
from jax import lax
from jax.experimental import pallas as pl
from jax.experimental.pallas import tpu as pltpu
from jax.experimental.pallas import tpu_sc as plsc

_N = 10000
_E = 160000
_LAT = 128
_NC, _NS = 2, 16
_NW = _NC * _NS
_ROWS_W = _E // _NW
_CHUNK = 200
_NCHUNK = _ROWS_W // _CHUNK
_NODE_W = _N // _NS
_BN = 1000
_BE = 2000
_F32 = jnp.float32


def _sc_mesh():
    return plsc.VectorSubcoreMesh(core_axis_name="c", subcore_axis_name="s",
                                  num_cores=_NC, num_subcores=_NS)


def _sc_gather(nlS, nlR, snd, rcv, lastpos=None):
    with_pos = lastpos is not None
    out_type = [jax.ShapeDtypeStruct((_E, _LAT), _F32),
                jax.ShapeDtypeStruct((_E, _LAT), _F32)]
    scratch = [pltpu.VMEM((_ROWS_W,), jnp.int32),
               pltpu.VMEM((_ROWS_W,), jnp.int32)]
    nbuf = 2
    per_buf = [pltpu.VMEM((_CHUNK, _LAT), _F32),
               pltpu.VMEM((_CHUNK, _LAT), _F32)]
    if with_pos:
        out_type += [jax.ShapeDtypeStruct((_E, _LAT), _F32),
                     jax.ShapeDtypeStruct((_E, _LAT), _F32)]
        per_buf += [pltpu.VMEM((_CHUNK, 16), _F32),
                    pltpu.VMEM((_CHUNK, 16), _F32)]
    nstream = len(per_buf)
    scratch += per_buf * nbuf
    scratch += [pltpu.SemaphoreType.DMA] * (2 * nbuf)

    @functools.partial(pl.kernel, out_type=out_type, mesh=_sc_mesh(),
                       scratch_types=scratch,
                       compiler_params=pltpu.CompilerParams(
                           use_tc_tiling_on_sc=False))
    def k(*refs):
        if with_pos:
            (nlS_h, nlR_h, snd_h, rcv_h, lp_h, gS_h, gR_h, gLS_h, gLR_h,
             idxS, idxR, *rest) = refs
            tabs = (nlS_h, nlR_h, lp_h, lp_h)
            outs = (gS_h, gR_h, gLS_h, gLR_h)
        else:
            (nlS_h, nlR_h, snd_h, rcv_h, gS_h, gR_h, idxS, idxR,
             *rest) = refs
            tabs = (nlS_h, nlR_h)
            outs = (gS_h, gR_h)
        bufs = [rest[b * nstream:(b + 1) * nstream] for b in range(nbuf)]
        sems = rest[nbuf * nstream:]
        sem_g = sems[:nbuf]
        sem_w = sems[nbuf:]
        idxs = (idxS, idxR, idxS, idxR)

        wid = lax.axis_index("s") * _NC + lax.axis_index("c")
        base0 = wid * _ROWS_W
        pltpu.sync_copy(snd_h.at[pl.ds(base0, _ROWS_W)], idxS)
        pltpu.sync_copy(rcv_h.at[pl.ds(base0, _ROWS_W)], idxR)

        def g_descs(c, b):
            ds_i = pl.ds(c * _CHUNK, _CHUNK)
            return [pltpu.make_async_copy(tabs[j].at[idxs[j].at[ds_i]],
                                          bufs[b][j], sem_g[b])
                    for j in range(nstream)]

        def w_descs(c, b):
            out_sl = pl.ds(base0 + c * _CHUNK, _CHUNK)
            ds = []
            for j in range(nstream):
                dst = (outs[j].at[out_sl] if j < 2
                       else outs[j].at[out_sl, pl.ds(0, 16)])
                ds.append(pltpu.make_async_copy(bufs[b][j], dst, sem_w[b]))
            return ds

        def fire_g(c, b):
            for d in g_descs(c, b):
                d.start()

        fire_g(0, 0)
        fire_g(1, 1)

        def step(c, b):
            for d in g_descs(c, b):
                d.wait()
            for d in w_descs(c, b):
                d.start()
            for d in w_descs(c, b):
                d.wait()

            @pl.when(c + nbuf < _NCHUNK)
            def _():
                fire_g(c + nbuf, b)

        @pl.loop(0, _NCHUNK // nbuf)
        def _(cc):
            step(cc * nbuf, 0)
            step(cc * nbuf + 1, 1)

        if _NCHUNK % nbuf:
            step(_NCHUNK - 1, (_NCHUNK - 1) % nbuf)

    if with_pos:
        return k(nlS, nlR, snd, rcv, lastpos)
    return k(nlS, nlR, snd, rcv)


def _sc_segment_sum(e_lat, rcv, zeros):
    nring = 4
    chunk = 40
    nchunk = _ROWS_W // chunk

    @functools.partial(
        pl.kernel,
        out_type=jax.ShapeDtypeStruct((_NC, _N, _LAT), _F32),
        mesh=_sc_mesh(),
        scratch_types=[pltpu.VMEM_SHARED((_N, _LAT), _F32)]
        + [pltpu.VMEM((chunk,), jnp.int32)] * nring
        + [pltpu.VMEM((chunk, _LAT), _F32)] * nring
        + [pltpu.SemaphoreType.DMA] * (2 * nring))
    def k(e_h, rcv_h, z_h, out_h, acc_sh, *rest):
        idxs = rest[:nring]
        rows = rest[nring:2 * nring]
        semL = rest[2 * nring:3 * nring]
        semA = rest[3 * nring:]
        cid = lax.axis_index("c")
        sid = lax.axis_index("s")
        pltpu.sync_copy(z_h.at[pl.ds(sid * 624, 624)],
                        acc_sh.at[pl.ds(sid * 624, 624)])

        @pl.when(sid == _NS - 1)
        def _():
            pltpu.sync_copy(z_h.at[pl.ds(_NS * 624, _N - _NS * 624)],
                            acc_sh.at[pl.ds(_NS * 624, _N - _NS * 624)])

        plsc.subcore_barrier()
        base0 = cid * (_E // _NC) + sid * (_E // _NC // _NS)

        def load_descs(c, b):
            base = base0 + c * chunk
            return [pltpu.make_async_copy(rcv_h.at[pl.ds(base, chunk)],
                                          idxs[b], semL[b]),
                    pltpu.make_async_copy(e_h.at[pl.ds(base, chunk)],
                                          rows[b], semL[b])]

        def add_desc(b):
            return pltpu.make_async_copy(rows[b], acc_sh.at[idxs[b]],
                                         semA[b])

        def fire_loads(c, b):
            for d in load_descs(c, b):
                d.start()

        for b in range(nring):
            fire_loads(b, b)

        def step(c, b):
            for d in load_descs(c, b):
                d.wait()
            add_desc(b).start(add=True)
            add_desc(b).wait()

            @pl.when(c + nring < nchunk)
            def _():
                fire_loads(c + nring, b)

        @pl.loop(0, nchunk // nring)
        def _(cc):
            for b in range(nring):
                step(cc * nring + b, b)

        for c in range((nchunk // nring) * nring, nchunk):
            step(c, c % nring)

        plsc.subcore_barrier()
        pltpu.sync_copy(acc_sh.at[pl.ds(sid * 624, 624)],
                        out_h.at[cid, pl.ds(sid * 624, 624)])

        @pl.when(sid == _NS - 1)
        def _():
            pltpu.sync_copy(acc_sh.at[pl.ds(_NS * 624, _N - _NS * 624)],
                            out_h.at[cid, pl.ds(_NS * 624, _N - _NS * 624)])

    return k(e_lat, rcv, zeros)


def _dot(a, b):
    return jnp.dot(a, b, preferred_element_type=_F32)


def _w_spec():
    return pl.BlockSpec((_LAT, _LAT), lambda i: (0, 0))


def _b_spec():
    return pl.BlockSpec((_LAT,), lambda i: (0,))


def _tc_node_encode(pos_flat, types_b, emb_table, Wv, We, b_enc, Wm2, Wm3):
    def body(pos_ref, t_ref, emb_ref, Wv_ref, We_ref, b_ref, W2_ref, W3_ref,
             nl_ref, s_ref, r_ref):
        pos = pos_ref[...]
        vel = pos[:, 2:12] - pos[:, 0:10]
        emb_proj = _dot(emb_ref[...], We_ref[...])
        t = t_ref[...]
        pre = _dot(vel, Wv_ref[...]) + b_ref[...][None, :]
        for kk in range(9):
            pre = pre + jnp.where(t == kk, emb_proj[kk][None, :], 0.0)
        nl = jnp.maximum(pre, 0.0)
        nl_ref[...] = nl
        s_ref[...] = _dot(nl, W2_ref[...])
        r_ref[...] = _dot(nl, W3_ref[...])

    grid = (_N // _BN,)
    return pl.pallas_call(
        body,
        grid=grid,
        in_specs=[
            pl.BlockSpec((_BN, 12), lambda i: (i, 0)),
            pl.BlockSpec((_BN, _LAT), lambda i: (i, 0)),
            pl.BlockSpec((9, 16), lambda i: (0, 0)),
            pl.BlockSpec((10, _LAT), lambda i: (0, 0)),
            pl.BlockSpec((16, _LAT), lambda i: (0, 0)),
            _b_spec(),
            _w_spec(),
            _w_spec(),
        ],
        out_specs=[pl.BlockSpec((_BN, _LAT), lambda i: (i, 0))] * 3,
        out_shape=[jax.ShapeDtypeStruct((_N, _LAT), _F32)] * 3,
    )(pos_flat, types_b, emb_table, Wv, We, b_enc, Wm2, Wm3)


def _tc_edge_step1(gS, gR, gLS, gLR, W_edge_enc, b_edge_enc, Wm1, b_msg):
    def body(gS_ref, gR_ref, gLS_ref, gLR_ref, We_ref, be_ref, W1_ref, bm_ref,
             out_ref):
        x = gLS_ref[:, 0:1] - gLR_ref[:, 0:1]
        y = gLS_ref[:, 1:2] - gLR_ref[:, 1:2]
        d = jnp.sqrt(x * x + y * y)
        We = We_ref[...]
        e0 = jnp.maximum(x * We[0][None, :] + y * We[1][None, :]
                         + d * We[2][None, :] + be_ref[...][None, :], 0.0)
        pre = _dot(e0, W1_ref[...]) + bm_ref[...][None, :] \
            + gS_ref[...] + gR_ref[...]
        out_ref[...] = e0 + jnp.maximum(pre, 0.0)

    grid = (_E // _BE,)
    return pl.pallas_call(
        body,
        grid=grid,
        in_specs=[
            pl.BlockSpec((_BE, _LAT), lambda i: (i, 0)),
            pl.BlockSpec((_BE, _LAT), lambda i: (i, 0)),
            pl.BlockSpec((_BE, _LAT), lambda i: (i, 0)),
            pl.BlockSpec((_BE, _LAT), lambda i: (i, 0)),
            pl.BlockSpec((3, _LAT), lambda i: (0, 0)),
            _b_spec(),
            _w_spec(),
            _b_spec(),
        ],
        out_specs=pl.BlockSpec((_BE, _LAT), lambda i: (i, 0)),
        out_shape=jax.ShapeDtypeStruct((_E, _LAT), _F32),
    )(gS, gR, gLS, gLR, W_edge_enc, b_edge_enc, Wm1, b_msg)


def _tc_edge_step2(e_prev, gS, gR, Wm1, b_msg):
    def body(e_ref, gS_ref, gR_ref, W1_ref, bm_ref, out_ref):
        e = e_ref[...]
        pre = _dot(e, W1_ref[...]) + bm_ref[...][None, :] \
            + gS_ref[...] + gR_ref[...]
        out_ref[...] = e + jnp.maximum(pre, 0.0)

    grid = (_E // _BE,)
    return pl.pallas_call(
        body,
        grid=grid,
        in_specs=[
            pl.BlockSpec((_BE, _LAT), lambda i: (i, 0)),
            pl.BlockSpec((_BE, _LAT), lambda i: (i, 0)),
            pl.BlockSpec((_BE, _LAT), lambda i: (i, 0)),
            _w_spec(),
            _b_spec(),
        ],
        out_specs=pl.BlockSpec((_BE, _LAT), lambda i: (i, 0)),
        out_shape=jax.ShapeDtypeStruct((_E, _LAT), _F32),
    )(e_prev, gS, gR, Wm1, b_msg)


def _tc_node_update(node_lat, partials, Wu1, Wu2, b_upd, Wm2, Wm3):
    def body(nl_ref, p_ref, Wu1_ref, Wu2_ref, bu_ref, W2_ref, W3_ref,
             nl2_ref, s_ref, r_ref):
        nl = nl_ref[...]
        agg = p_ref[0] + p_ref[1]
        nl2 = nl + jnp.maximum(
            _dot(nl, Wu1_ref[...]) + _dot(agg, Wu2_ref[...])
            + bu_ref[...][None, :], 0.0)
        nl2_ref[...] = nl2
        s_ref[...] = _dot(nl2, W2_ref[...])
        r_ref[...] = _dot(nl2, W3_ref[...])

    grid = (_N // _BN,)
    return pl.pallas_call(
        body,
        grid=grid,
        in_specs=[
            pl.BlockSpec((_BN, _LAT), lambda i: (i, 0)),
            pl.BlockSpec((_NC, _BN, _LAT), lambda i: (0, i, 0)),
            _w_spec(), _w_spec(), _b_spec(), _w_spec(), _w_spec(),
        ],
        out_specs=[pl.BlockSpec((_BN, _LAT), lambda i: (i, 0))] * 3,
        out_shape=[jax.ShapeDtypeStruct((_N, _LAT), _F32)] * 3,
    )(node_lat, partials, Wu1, Wu2, b_upd, Wm2, Wm3)


def _tc_node_final(node_lat, partials, Wu1, Wu2, b_upd, Wd_pad, bd_pad):
    def body(nl_ref, p_ref, Wu1_ref, Wu2_ref, bu_ref, Wd_ref, bd_ref,
             out_ref):
        nl = nl_ref[...]
        agg = p_ref[0] + p_ref[1]
        nl2 = nl + jnp.maximum(
            _dot(nl, Wu1_ref[...]) + _dot(agg, Wu2_ref[...])
            + bu_ref[...][None, :], 0.0)
        out_ref[...] = _dot(nl2, Wd_ref[...]) + bd_ref[...][None, :]

    grid = (_N // _BN,)
    return pl.pallas_call(
        body,
        grid=grid,
        in_specs=[
            pl.BlockSpec((_BN, _LAT), lambda i: (i, 0)),
            pl.BlockSpec((_NC, _BN, _LAT), lambda i: (0, i, 0)),
            _w_spec(), _w_spec(), _b_spec(), _w_spec(), _b_spec(),
        ],
        out_specs=pl.BlockSpec((_BN, _LAT), lambda i: (i, 0)),
        out_shape=jax.ShapeDtypeStruct((_N, _LAT), _F32),
    )(node_lat, partials, Wu1, Wu2, b_upd, Wd_pad, bd_pad)


def kernel(position_sequence, particle_types, edge_index, emb_table,
           W_node_enc, b_node_enc, W_edge_enc, b_edge_enc,
           W_msg, b_msg, W_upd, b_upd, W_dec, b_dec):
    pos_flat = position_sequence.reshape(_N, 12)
    types_b = jnp.broadcast_to(
        particle_types.astype(jnp.int32)[:, None], (_N, _LAT))
    snd = edge_index[0].astype(jnp.int32)
    rcv = edge_index[1].astype(jnp.int32)
    lastpos = jnp.pad(pos_flat[:, 10:12], ((0, 0), (0, 14)))
    Wv = W_node_enc[:10]
    We = W_node_enc[10:]
    Wm1 = W_msg[:_LAT]
    Wm2 = W_msg[_LAT:2 * _LAT]
    Wm3 = W_msg[2 * _LAT:]
    Wu1 = W_upd[:_LAT]
    Wu2 = W_upd[_LAT:]
    Wd_pad = jnp.pad(W_dec, ((0, 0), (0, _LAT - W_dec.shape[1])))
    bd_pad = jnp.pad(b_dec, (0, _LAT - b_dec.shape[0]))
    zeros = jnp.zeros((_N, _LAT), _F32)

    nl0, nlS0, nlR0 = _tc_node_encode(
        pos_flat, types_b, emb_table, Wv, We, b_node_enc, Wm2, Wm3)

    gS1, gR1, gLS, gLR = _sc_gather(nlS0, nlR0, snd, rcv, lastpos)
    e1 = _tc_edge_step1(gS1, gR1, gLS, gLR, W_edge_enc, b_edge_enc,
                        Wm1, b_msg)
    p1 = _sc_segment_sum(e1, rcv, zeros)
    nl1, nlS1, nlR1 = _tc_node_update(nl0, p1, Wu1, Wu2, b_upd, Wm2, Wm3)

    gS2, gR2 = _sc_gather(nlS1, nlR1, snd, rcv)
    e2 = _tc_edge_step2(e1, gS2, gR2, Wm1, b_msg)
    p2 = _sc_segment_sum(e2, rcv, zeros)
    out_pad = _tc_node_final(nl1, p2, Wu1, Wu2, b_upd, Wd_pad, bd_pad)

    return out_pad[:, :W_dec.shape[1]]


# speedup vs baseline: 5.1925x; 1.0131x over previous
import functools

import jax
import jax.numpy as jnp
from jax import lax
from jax.experimental import pallas as pl
from jax.experimental.pallas import tpu as pltpu
from jax.experimental.pallas import tpu_sc as plsc

_N = 10000
_E = 160000
_LAT = 128
_NC, _NS = 2, 16
_NW = _NC * _NS
_CHUNK = 200
_H0 = 83200
_H1 = _E - _H0
_BN = 1000
_BE = 1600
_F32 = jnp.float32


def _sc_mesh():
    return plsc.VectorSubcoreMesh(core_axis_name="c", subcore_axis_name="s",
                                  num_cores=_NC, num_subcores=_NS)


def _sc_gather(nlS, nlR, snd, rcv, start, count, lastpos=None):
    with_pos = lastpos is not None
    per_tile = count // _NW
    nchunk = per_tile // _CHUNK
    assert per_tile % _CHUNK == 0 and per_tile % 8 == 0
    out_type = [jax.ShapeDtypeStruct((count, _LAT), _F32),
                jax.ShapeDtypeStruct((count, _LAT), _F32)]
    scratch = [pltpu.VMEM((per_tile,), jnp.int32),
               pltpu.VMEM((per_tile,), jnp.int32)]
    nbuf = 2
    per_buf = [pltpu.VMEM((_CHUNK, _LAT), _F32),
               pltpu.VMEM((_CHUNK, _LAT), _F32)]
    if with_pos:
        out_type += [jax.ShapeDtypeStruct((count, _LAT), _F32),
                     jax.ShapeDtypeStruct((count, _LAT), _F32)]
        per_buf += [pltpu.VMEM((_CHUNK, 16), _F32),
                    pltpu.VMEM((_CHUNK, 16), _F32)]
    nstream = len(per_buf)
    scratch += per_buf * nbuf
    scratch += [pltpu.SemaphoreType.DMA] * (2 * nbuf)

    @functools.partial(pl.kernel, out_type=out_type, mesh=_sc_mesh(),
                       scratch_types=scratch,
                       compiler_params=pltpu.CompilerParams(
                           use_tc_tiling_on_sc=False))
    def k(*refs):
        if with_pos:
            (nlS_h, nlR_h, snd_h, rcv_h, lp_h, gS_h, gR_h, gLS_h, gLR_h,
             idxS, idxR, *rest) = refs
            tabs = (nlS_h, nlR_h, lp_h, lp_h)
            outs = (gS_h, gR_h, gLS_h, gLR_h)
        else:
            (nlS_h, nlR_h, snd_h, rcv_h, gS_h, gR_h, idxS, idxR,
             *rest) = refs
            tabs = (nlS_h, nlR_h)
            outs = (gS_h, gR_h)
        bufs = [rest[b * nstream:(b + 1) * nstream] for b in range(nbuf)]
        sems = rest[nbuf * nstream:]
        sem_g = sems[:nbuf]
        sem_w = sems[nbuf:]
        idxs = (idxS, idxR, idxS, idxR)

        wid = lax.axis_index("s") * _NC + lax.axis_index("c")
        tile0 = wid * per_tile
        pltpu.sync_copy(snd_h.at[pl.ds(start + tile0, per_tile)], idxS)
        pltpu.sync_copy(rcv_h.at[pl.ds(start + tile0, per_tile)], idxR)

        def g_descs(c, b):
            ds_i = pl.ds(c * _CHUNK, _CHUNK)
            return [pltpu.make_async_copy(tabs[j].at[idxs[j].at[ds_i]],
                                          bufs[b][j], sem_g[b])
                    for j in range(nstream)]

        def w_descs(c, b):
            out_sl = pl.ds(tile0 + c * _CHUNK, _CHUNK)
            ds = []
            for j in range(nstream):
                dst = (outs[j].at[out_sl] if j < 2
                       else outs[j].at[out_sl, pl.ds(0, 16)])
                ds.append(pltpu.make_async_copy(bufs[b][j], dst, sem_w[b]))
            return ds

        def fire_g(c, b):
            for d in g_descs(c, b):
                d.start()

        fire_g(0, 0)
        fire_g(1, 1)

        def step(c, b):
            for d in g_descs(c, b):
                d.wait()
            for d in w_descs(c, b):
                d.start()
            for d in w_descs(c, b):
                d.wait()

            @pl.when(c + nbuf < nchunk)
            def _():
                fire_g(c + nbuf, b)

        @pl.loop(0, nchunk // nbuf)
        def _(cc):
            step(cc * nbuf, 0)
            step(cc * nbuf + 1, 1)

        if nchunk % nbuf:
            step(nchunk - 1, (nchunk - 1) % nbuf)

    if with_pos:
        return k(nlS, nlR, snd, rcv, lastpos)
    return k(nlS, nlR, snd, rcv)


def _sc_segment_sum(e_lat, rcv, zeros, start, count):
    nring = 4
    chunk = 40
    per_tile = count // _NW
    nchunk = per_tile // chunk
    assert per_tile % chunk == 0 and per_tile % 8 == 0

    @functools.partial(
        pl.kernel,
        out_type=jax.ShapeDtypeStruct((_NC, _N, _LAT), _F32),
        mesh=_sc_mesh(),
        scratch_types=[pltpu.VMEM_SHARED((_N, _LAT), _F32)]
        + [pltpu.VMEM((chunk,), jnp.int32)] * nring
        + [pltpu.VMEM((chunk, _LAT), _F32)] * nring
        + [pltpu.SemaphoreType.DMA] * (2 * nring))
    def k(e_h, rcv_h, z_h, out_h, acc_sh, *rest):
        idxs = rest[:nring]
        rows = rest[nring:2 * nring]
        semL = rest[2 * nring:3 * nring]
        semA = rest[3 * nring:]
        cid = lax.axis_index("c")
        sid = lax.axis_index("s")
        pltpu.sync_copy(z_h.at[pl.ds(sid * 624, 624)],
                        acc_sh.at[pl.ds(sid * 624, 624)])

        @pl.when(sid == _NS - 1)
        def _():
            pltpu.sync_copy(z_h.at[pl.ds(_NS * 624, _N - _NS * 624)],
                            acc_sh.at[pl.ds(_NS * 624, _N - _NS * 624)])

        plsc.subcore_barrier()
        base0 = cid * (count // _NC) + sid * per_tile

        def load_descs(c, b):
            base = base0 + c * chunk
            return [pltpu.make_async_copy(rcv_h.at[pl.ds(start + base,
                                                         chunk)],
                                          idxs[b], semL[b]),
                    pltpu.make_async_copy(e_h.at[pl.ds(base, chunk)],
                                          rows[b], semL[b])]

        def add_desc(b):
            return pltpu.make_async_copy(rows[b], acc_sh.at[idxs[b]],
                                         semA[b])

        def fire_loads(c, b):
            for d in load_descs(c, b):
                d.start()

        for b in range(nring):
            fire_loads(b, b)

        def step(c, b):
            for d in load_descs(c, b):
                d.wait()
            add_desc(b).start(add=True)
            add_desc(b).wait()

            @pl.when(c + nring < nchunk)
            def _():
                fire_loads(c + nring, b)

        @pl.loop(0, nchunk // nring)
        def _(cc):
            for b in range(nring):
                step(cc * nring + b, b)

        for c in range((nchunk // nring) * nring, nchunk):
            step(c, c % nring)

        plsc.subcore_barrier()
        pltpu.sync_copy(acc_sh.at[pl.ds(sid * 624, 624)],
                        out_h.at[cid, pl.ds(sid * 624, 624)])

        @pl.when(sid == _NS - 1)
        def _():
            pltpu.sync_copy(acc_sh.at[pl.ds(_NS * 624, _N - _NS * 624)],
                            out_h.at[cid, pl.ds(_NS * 624, _N - _NS * 624)])

    return k(e_lat, rcv, zeros)


def _dot(a, b):
    return jnp.dot(a, b, preferred_element_type=_F32)


def _w_spec():
    return pl.BlockSpec((_LAT, _LAT), lambda i: (0, 0))


def _b_spec():
    return pl.BlockSpec((_LAT,), lambda i: (0,))


def _e_spec():
    return pl.BlockSpec((_BE, _LAT), lambda i: (i, 0))


def _tc_node_encode(pos_flat, types_b, emb_table, Wv, We, b_enc, Wm2, Wm3):
    def body(pos_ref, t_ref, emb_ref, Wv_ref, We_ref, b_ref, W2_ref, W3_ref,
             nl_ref, s_ref, r_ref):
        pos = pos_ref[...]
        vel = pos[:, 2:12] - pos[:, 0:10]
        emb_proj = _dot(emb_ref[...], We_ref[...])
        t = t_ref[...]
        pre = _dot(vel, Wv_ref[...]) + b_ref[...][None, :]
        for kk in range(9):
            pre = pre + jnp.where(t == kk, emb_proj[kk][None, :], 0.0)
        nl = jnp.maximum(pre, 0.0)
        nl_ref[...] = nl
        s_ref[...] = _dot(nl, W2_ref[...])
        r_ref[...] = _dot(nl, W3_ref[...])

    return pl.pallas_call(
        body,
        grid=(_N // _BN,),
        in_specs=[
            pl.BlockSpec((_BN, 12), lambda i: (i, 0)),
            pl.BlockSpec((_BN, _LAT), lambda i: (i, 0)),
            pl.BlockSpec((9, 16), lambda i: (0, 0)),
            pl.BlockSpec((10, _LAT), lambda i: (0, 0)),
            pl.BlockSpec((16, _LAT), lambda i: (0, 0)),
            _b_spec(),
            _w_spec(),
            _w_spec(),
        ],
        out_specs=[pl.BlockSpec((_BN, _LAT), lambda i: (i, 0))] * 3,
        out_shape=[jax.ShapeDtypeStruct((_N, _LAT), _F32)] * 3,
    )(pos_flat, types_b, emb_table, Wv, We, b_enc, Wm2, Wm3)


def _tc_edge_step1(gS, gR, gLS, gLR, W_edge_enc, b_edge_enc, Wm1, b_msg):
    count = gS.shape[0]

    def body(gS_ref, gR_ref, gLS_ref, gLR_ref, We_ref, be_ref, W1_ref, bm_ref,
             out_ref):
        x = gLS_ref[:, 0:1] - gLR_ref[:, 0:1]
        y = gLS_ref[:, 1:2] - gLR_ref[:, 1:2]
        d = jnp.sqrt(x * x + y * y)
        We = We_ref[...]
        e0 = jnp.maximum(x * We[0][None, :] + y * We[1][None, :]
                         + d * We[2][None, :] + be_ref[...][None, :], 0.0)
        pre = _dot(e0, W1_ref[...]) + bm_ref[...][None, :] \
            + gS_ref[...] + gR_ref[...]
        out_ref[...] = e0 + jnp.maximum(pre, 0.0)

    return pl.pallas_call(
        body,
        grid=(count // _BE,),
        in_specs=[
            _e_spec(),
            _e_spec(),
            _e_spec(),
            _e_spec(),
            pl.BlockSpec((3, _LAT), lambda i: (0, 0)),
            _b_spec(),
            _w_spec(),
            _b_spec(),
        ],
        out_specs=_e_spec(),
        out_shape=jax.ShapeDtypeStruct((count, _LAT), _F32),
    )(gS, gR, gLS, gLR, W_edge_enc, b_edge_enc, Wm1, b_msg)


def _tc_edge_step2(e_prev, gS, gR, Wm1, b_msg):
    count = gS.shape[0]

    def body(e_ref, gS_ref, gR_ref, W1_ref, bm_ref, out_ref):
        e = e_ref[...]
        pre = _dot(e, W1_ref[...]) + bm_ref[...][None, :] \
            + gS_ref[...] + gR_ref[...]
        out_ref[...] = e + jnp.maximum(pre, 0.0)

    return pl.pallas_call(
        body,
        grid=(count // _BE,),
        in_specs=[_e_spec(), _e_spec(), _e_spec(), _w_spec(), _b_spec()],
        out_specs=_e_spec(),
        out_shape=jax.ShapeDtypeStruct((count, _LAT), _F32),
    )(e_prev, gS, gR, Wm1, b_msg)


def _p_spec():
    return pl.BlockSpec((_NC, _BN, _LAT), lambda i: (0, i, 0))


def _tc_node_update(node_lat, pa, pb, Wu1, Wu2, b_upd, Wm2, Wm3):
    def body(nl_ref, pa_ref, pb_ref, Wu1_ref, Wu2_ref, bu_ref, W2_ref,
             W3_ref, nl2_ref, s_ref, r_ref):
        nl = nl_ref[...]
        agg = pa_ref[0] + pa_ref[1] + pb_ref[0] + pb_ref[1]
        nl2 = nl + jnp.maximum(
            _dot(nl, Wu1_ref[...]) + _dot(agg, Wu2_ref[...])
            + bu_ref[...][None, :], 0.0)
        nl2_ref[...] = nl2
        s_ref[...] = _dot(nl2, W2_ref[...])
        r_ref[...] = _dot(nl2, W3_ref[...])

    return pl.pallas_call(
        body,
        grid=(_N // _BN,),
        in_specs=[
            pl.BlockSpec((_BN, _LAT), lambda i: (i, 0)),
            _p_spec(), _p_spec(),
            _w_spec(), _w_spec(), _b_spec(), _w_spec(), _w_spec(),
        ],
        out_specs=[pl.BlockSpec((_BN, _LAT), lambda i: (i, 0))] * 3,
        out_shape=[jax.ShapeDtypeStruct((_N, _LAT), _F32)] * 3,
    )(node_lat, pa, pb, Wu1, Wu2, b_upd, Wm2, Wm3)


def _tc_node_final(node_lat, pa, pb, Wu1, Wu2, b_upd, Wd_pad, bd_pad):
    def body(nl_ref, pa_ref, pb_ref, Wu1_ref, Wu2_ref, bu_ref, Wd_ref,
             bd_ref, out_ref):
        nl = nl_ref[...]
        agg = pa_ref[0] + pa_ref[1] + pb_ref[0] + pb_ref[1]
        nl2 = nl + jnp.maximum(
            _dot(nl, Wu1_ref[...]) + _dot(agg, Wu2_ref[...])
            + bu_ref[...][None, :], 0.0)
        out_ref[...] = _dot(nl2, Wd_ref[...]) + bd_ref[...][None, :]

    return pl.pallas_call(
        body,
        grid=(_N // _BN,),
        in_specs=[
            pl.BlockSpec((_BN, _LAT), lambda i: (i, 0)),
            _p_spec(), _p_spec(),
            _w_spec(), _w_spec(), _b_spec(), _w_spec(), _b_spec(),
        ],
        out_specs=pl.BlockSpec((_BN, _LAT), lambda i: (i, 0)),
        out_shape=jax.ShapeDtypeStruct((_N, _LAT), _F32),
    )(node_lat, pa, pb, Wu1, Wu2, b_upd, Wd_pad, bd_pad)


def kernel(position_sequence, particle_types, edge_index, emb_table,
           W_node_enc, b_node_enc, W_edge_enc, b_edge_enc,
           W_msg, b_msg, W_upd, b_upd, W_dec, b_dec):
    pos_flat = position_sequence.reshape(_N, 12)
    types_b = jnp.broadcast_to(
        particle_types.astype(jnp.int32)[:, None], (_N, _LAT))
    snd = edge_index[0].astype(jnp.int32)
    rcv = edge_index[1].astype(jnp.int32)
    lastpos = jnp.pad(pos_flat[:, 10:12], ((0, 0), (0, 14)))
    Wv = W_node_enc[:10]
    We = W_node_enc[10:]
    Wm1 = W_msg[:_LAT]
    Wm2 = W_msg[_LAT:2 * _LAT]
    Wm3 = W_msg[2 * _LAT:]
    Wu1 = W_upd[:_LAT]
    Wu2 = W_upd[_LAT:]
    Wd_pad = jnp.pad(W_dec, ((0, 0), (0, _LAT - W_dec.shape[1])))
    bd_pad = jnp.pad(b_dec, (0, _LAT - b_dec.shape[0]))
    zeros = jnp.zeros((_N, _LAT), _F32)

    nl0, nlS0, nlR0 = _tc_node_encode(
        pos_flat, types_b, emb_table, Wv, We, b_node_enc, Wm2, Wm3)

    ga = _sc_gather(nlS0, nlR0, snd, rcv, 0, _H0, lastpos)
    gb = _sc_gather(nlS0, nlR0, snd, rcv, _H0, _H1, lastpos)
    e1a = _tc_edge_step1(*ga, W_edge_enc, b_edge_enc, Wm1, b_msg)
    e1b = _tc_edge_step1(*gb, W_edge_enc, b_edge_enc, Wm1, b_msg)
    pa1 = _sc_segment_sum(e1a, rcv, zeros, 0, _H0)
    pb1 = _sc_segment_sum(e1b, rcv, zeros, _H0, _H1)
    nl1, nlS1, nlR1 = _tc_node_update(nl0, pa1, pb1, Wu1, Wu2, b_upd,
                                      Wm2, Wm3)

    gS2a, gR2a = _sc_gather(nlS1, nlR1, snd, rcv, 0, _H0)
    gS2b, gR2b = _sc_gather(nlS1, nlR1, snd, rcv, _H0, _H1)
    e2a = _tc_edge_step2(e1a, gS2a, gR2a, Wm1, b_msg)
    e2b = _tc_edge_step2(e1b, gS2b, gR2b, Wm1, b_msg)
    pa2 = _sc_segment_sum(e2a, rcv, zeros, 0, _H0)
    pb2 = _sc_segment_sum(e2b, rcv, zeros, _H0, _H1)
    out_pad = _tc_node_final(nl1, pa2, pb2, Wu1, Wu2, b_upd, Wd_pad, bd_pad)

    return out_pad[:, :W_dec.shape[1]]


# speedup vs baseline: 5.5144x; 1.0620x over previous
import functools

import jax
import jax.numpy as jnp
from jax import lax
from jax.experimental import pallas as pl
from jax.experimental.pallas import tpu as pltpu
from jax.experimental.pallas import tpu_sc as plsc

_N = 10000
_E = 160000
_LAT = 128
_NC, _NS = 2, 16
_NW = _NC * _NS
_CHUNK = 200
_H0 = 83200
_H1 = _E - _H0
_BN = 1000
_BE = 1600
_F32 = jnp.float32


def _sc_mesh():
    return plsc.VectorSubcoreMesh(core_axis_name="c", subcore_axis_name="s",
                                  num_cores=_NC, num_subcores=_NS)


def _sc_gather(nlS, nlR, snd, rcv, start, count, lastpos=None):
    with_pos = lastpos is not None
    per_tile = count // _NW
    nchunk = per_tile // _CHUNK
    assert per_tile % _CHUNK == 0 and per_tile % 8 == 0
    out_type = [jax.ShapeDtypeStruct((count, _LAT), _F32),
                jax.ShapeDtypeStruct((count, _LAT), _F32)]
    scratch = [pltpu.VMEM((per_tile,), jnp.int32),
               pltpu.VMEM((per_tile,), jnp.int32)]
    nbuf = 2
    per_buf = [pltpu.VMEM((_CHUNK, _LAT), _F32),
               pltpu.VMEM((_CHUNK, _LAT), _F32)]
    if with_pos:
        out_type += [jax.ShapeDtypeStruct((count, _LAT), _F32)]
        per_buf += [pltpu.VMEM((_CHUNK, 16), _F32),
                    pltpu.VMEM((_CHUNK, 16), _F32)]
    nstream = len(per_buf)
    scratch += per_buf * nbuf
    scratch += [pltpu.SemaphoreType.DMA] * (2 * nbuf)

    @functools.partial(pl.kernel, out_type=out_type, mesh=_sc_mesh(),
                       scratch_types=scratch,
                       compiler_params=pltpu.CompilerParams(
                           use_tc_tiling_on_sc=False))
    def k(*refs):
        if with_pos:
            (nlS_h, nlR_h, snd_h, rcv_h, lp_h, gS_h, gR_h, rel_h,
             idxS, idxR, *rest) = refs
            tabs = (nlS_h, nlR_h, lp_h, lp_h)
            outs = (gS_h, gR_h, rel_h)
        else:
            (nlS_h, nlR_h, snd_h, rcv_h, gS_h, gR_h, idxS, idxR,
             *rest) = refs
            tabs = (nlS_h, nlR_h)
            outs = (gS_h, gR_h)
        bufs = [rest[b * nstream:(b + 1) * nstream] for b in range(nbuf)]
        sems = rest[nbuf * nstream:]
        sem_g = sems[:nbuf]
        sem_w = sems[nbuf:]
        idxs = (idxS, idxR, idxS, idxR)

        wid = lax.axis_index("s") * _NC + lax.axis_index("c")
        tile0 = wid * per_tile
        pltpu.sync_copy(snd_h.at[pl.ds(start + tile0, per_tile)], idxS)
        pltpu.sync_copy(rcv_h.at[pl.ds(start + tile0, per_tile)], idxR)

        def g_descs(c, b):
            ds_i = pl.ds(c * _CHUNK, _CHUNK)
            return [pltpu.make_async_copy(tabs[j].at[idxs[j].at[ds_i]],
                                          bufs[b][j], sem_g[b])
                    for j in range(nstream)]

        def w_descs(c, b):
            out_sl = pl.ds(tile0 + c * _CHUNK, _CHUNK)
            ds = []
            for j in range(len(outs)):
                dst = (outs[j].at[out_sl] if j < 2
                       else outs[j].at[out_sl, pl.ds(0, 16)])
                ds.append(pltpu.make_async_copy(bufs[b][j], dst, sem_w[b]))
            return ds

        def fire_g(c, b):
            for d in g_descs(c, b):
                d.start()

        fire_g(0, 0)
        fire_g(1, 1)

        def step(c, b):
            for d in g_descs(c, b):
                d.wait()
            if with_pos:
                ps, pr = bufs[b][2], bufs[b][3]

                @pl.loop(0, _CHUNK)
                def _(r):
                    ps[r] = ps[r] - pr[r]

            for d in w_descs(c, b):
                d.start()
            for d in w_descs(c, b):
                d.wait()

            @pl.when(c + nbuf < nchunk)
            def _():
                fire_g(c + nbuf, b)

        @pl.loop(0, nchunk // nbuf)
        def _(cc):
            step(cc * nbuf, 0)
            step(cc * nbuf + 1, 1)

        if nchunk % nbuf:
            step(nchunk - 1, (nchunk - 1) % nbuf)

    if with_pos:
        return k(nlS, nlR, snd, rcv, lastpos)
    return k(nlS, nlR, snd, rcv)


def _sc_segment_sum(e_lat, rcv, start, count):
    nring = 4
    chunk = 40
    per_tile = count // _NW
    nchunk = per_tile // chunk
    assert per_tile % chunk == 0 and per_tile % 8 == 0

    @functools.partial(
        pl.kernel,
        out_type=jax.ShapeDtypeStruct((_NC, _N, _LAT), _F32),
        mesh=_sc_mesh(),
        scratch_types=[pltpu.VMEM_SHARED((_N, _LAT), _F32)]
        + [pltpu.VMEM((chunk,), jnp.int32)] * nring
        + [pltpu.VMEM((chunk, _LAT), _F32)] * nring
        + [pltpu.SemaphoreType.DMA] * (2 * nring))
    def k(e_h, rcv_h, out_h, acc_sh, *rest):
        idxs = rest[:nring]
        rows = rest[nring:2 * nring]
        semL = rest[2 * nring:3 * nring]
        semA = rest[3 * nring:]
        cid = lax.axis_index("c")
        sid = lax.axis_index("s")
        zb = rows[0]

        @pl.loop(0, chunk)
        def _(r):
            @pl.loop(0, _LAT // 16)
            def _(l):
                zb[r, pl.ds(l * 16, 16)] = jnp.zeros((16,), _F32)

        @pl.loop(0, 624 // chunk)
        def _(kk):
            pltpu.sync_copy(zb, acc_sh.at[pl.ds(sid * 624 + kk * chunk,
                                                chunk)])
        pltpu.sync_copy(zb.at[pl.ds(0, 624 % chunk)],
                        acc_sh.at[pl.ds(sid * 624 + (624 // chunk) * chunk,
                                        624 % chunk)])

        @pl.when(sid == _NS - 1)
        def _():
            pltpu.sync_copy(zb.at[pl.ds(0, _N - _NS * 624)],
                            acc_sh.at[pl.ds(_NS * 624, _N - _NS * 624)])

        plsc.subcore_barrier()
        base0 = cid * (count // _NC) + sid * per_tile

        def load_descs(c, b):
            base = base0 + c * chunk
            return [pltpu.make_async_copy(rcv_h.at[pl.ds(start + base,
                                                         chunk)],
                                          idxs[b], semL[b]),
                    pltpu.make_async_copy(e_h.at[pl.ds(base, chunk)],
                                          rows[b], semL[b])]

        def add_desc(b):
            return pltpu.make_async_copy(rows[b], acc_sh.at[idxs[b]],
                                         semA[b])

        def fire_loads(c, b):
            for d in load_descs(c, b):
                d.start()

        for b in range(nring):
            fire_loads(b, b)

        def step(c, b):
            for d in load_descs(c, b):
                d.wait()
            add_desc(b).start(add=True)
            add_desc(b).wait()

            @pl.when(c + nring < nchunk)
            def _():
                fire_loads(c + nring, b)

        @pl.loop(0, nchunk // nring)
        def _(cc):
            for b in range(nring):
                step(cc * nring + b, b)

        for c in range((nchunk // nring) * nring, nchunk):
            step(c, c % nring)

        plsc.subcore_barrier()
        pltpu.sync_copy(acc_sh.at[pl.ds(sid * 624, 624)],
                        out_h.at[cid, pl.ds(sid * 624, 624)])

        @pl.when(sid == _NS - 1)
        def _():
            pltpu.sync_copy(acc_sh.at[pl.ds(_NS * 624, _N - _NS * 624)],
                            out_h.at[cid, pl.ds(_NS * 624, _N - _NS * 624)])

    return k(e_lat, rcv)


def _dot(a, b):
    return jnp.dot(a, b, preferred_element_type=_F32)


def _w_spec():
    return pl.BlockSpec((_LAT, _LAT), lambda i: (0, 0))


def _b_spec():
    return pl.BlockSpec((_LAT,), lambda i: (0,))


def _e_spec():
    return pl.BlockSpec((_BE, _LAT), lambda i: (i, 0))


def _tc_node_encode(pos_flat, types_b, emb_table, Wv, We, b_enc, Wm2, Wm3):
    def body(pos_ref, t_ref, emb_ref, Wv_ref, We_ref, b_ref, W2_ref, W3_ref,
             nl_ref, s_ref, r_ref):
        pos = pos_ref[...]
        vel = pos[:, 2:12] - pos[:, 0:10]
        emb_proj = _dot(emb_ref[...], We_ref[...])
        t = t_ref[...]
        pre = _dot(vel, Wv_ref[...]) + b_ref[...][None, :]
        for kk in range(9):
            pre = pre + jnp.where(t == kk, emb_proj[kk][None, :], 0.0)
        nl = jnp.maximum(pre, 0.0)
        nl_ref[...] = nl
        s_ref[...] = _dot(nl, W2_ref[...])
        r_ref[...] = _dot(nl, W3_ref[...])

    return pl.pallas_call(
        body,
        grid=(_N // _BN,),
        in_specs=[
            pl.BlockSpec((_BN, 12), lambda i: (i, 0)),
            pl.BlockSpec((_BN, _LAT), lambda i: (i, 0)),
            pl.BlockSpec((9, 16), lambda i: (0, 0)),
            pl.BlockSpec((10, _LAT), lambda i: (0, 0)),
            pl.BlockSpec((16, _LAT), lambda i: (0, 0)),
            _b_spec(),
            _w_spec(),
            _w_spec(),
        ],
        out_specs=[pl.BlockSpec((_BN, _LAT), lambda i: (i, 0))] * 3,
        out_shape=[jax.ShapeDtypeStruct((_N, _LAT), _F32)] * 3,
    )(pos_flat, types_b, emb_table, Wv, We, b_enc, Wm2, Wm3)


def _tc_edge_step1(gS, gR, rel, M3, b_edge_enc, Wm1, b_msg):
    count = gS.shape[0]

    def body(gS_ref, gR_ref, rel_ref, M3_ref, be_ref, W1_ref, bm_ref,
             out_ref):
        lane = lax.broadcasted_iota(jnp.int32, (_BE, _LAT), 1)
        X = jnp.where(lane < 2, rel_ref[...], 0.0)
        d = jnp.sqrt(X[:, 0:1] * X[:, 0:1] + X[:, 1:2] * X[:, 1:2])
        XD = jnp.where(lane == 2, d, X)
        e0 = jnp.maximum(_dot(XD, M3_ref[...]) + be_ref[...][None, :], 0.0)
        pre = _dot(e0, W1_ref[...]) + bm_ref[...][None, :] \
            + gS_ref[...] + gR_ref[...]
        out_ref[...] = e0 + jnp.maximum(pre, 0.0)

    return pl.pallas_call(
        body,
        grid=(count // _BE,),
        in_specs=[
            _e_spec(),
            _e_spec(),
            _e_spec(),
            _w_spec(),
            _b_spec(),
            _w_spec(),
            _b_spec(),
        ],
        out_specs=_e_spec(),
        out_shape=jax.ShapeDtypeStruct((count, _LAT), _F32),
    )(gS, gR, rel, M3, b_edge_enc, Wm1, b_msg)


def _tc_edge_step2(e_prev, gS, gR, Wm1, b_msg):
    count = gS.shape[0]

    def body(e_ref, gS_ref, gR_ref, W1_ref, bm_ref, out_ref):
        e = e_ref[...]
        pre = _dot(e, W1_ref[...]) + bm_ref[...][None, :] \
            + gS_ref[...] + gR_ref[...]
        out_ref[...] = e + jnp.maximum(pre, 0.0)

    return pl.pallas_call(
        body,
        grid=(count // _BE,),
        in_specs=[_e_spec(), _e_spec(), _e_spec(), _w_spec(), _b_spec()],
        out_specs=_e_spec(),
        out_shape=jax.ShapeDtypeStruct((count, _LAT), _F32),
    )(e_prev, gS, gR, Wm1, b_msg)


def _p_spec():
    return pl.BlockSpec((_NC, _BN, _LAT), lambda i: (0, i, 0))


def _tc_node_update(node_lat, pa, pb, Wu1, Wu2, b_upd, Wm2, Wm3):
    def body(nl_ref, pa_ref, pb_ref, Wu1_ref, Wu2_ref, bu_ref, W2_ref,
             W3_ref, nl2_ref, s_ref, r_ref):
        nl = nl_ref[...]
        agg = pa_ref[0] + pa_ref[1] + pb_ref[0] + pb_ref[1]
        nl2 = nl + jnp.maximum(
            _dot(nl, Wu1_ref[...]) + _dot(agg, Wu2_ref[...])
            + bu_ref[...][None, :], 0.0)
        nl2_ref[...] = nl2
        s_ref[...] = _dot(nl2, W2_ref[...])
        r_ref[...] = _dot(nl2, W3_ref[...])

    return pl.pallas_call(
        body,
        grid=(_N // _BN,),
        in_specs=[
            pl.BlockSpec((_BN, _LAT), lambda i: (i, 0)),
            _p_spec(), _p_spec(),
            _w_spec(), _w_spec(), _b_spec(), _w_spec(), _w_spec(),
        ],
        out_specs=[pl.BlockSpec((_BN, _LAT), lambda i: (i, 0))] * 3,
        out_shape=[jax.ShapeDtypeStruct((_N, _LAT), _F32)] * 3,
    )(node_lat, pa, pb, Wu1, Wu2, b_upd, Wm2, Wm3)


def _tc_node_final(node_lat, pa, pb, Wu1, Wu2, b_upd, Wd_pad, bd_pad):
    def body(nl_ref, pa_ref, pb_ref, Wu1_ref, Wu2_ref, bu_ref, Wd_ref,
             bd_ref, out_ref):
        nl = nl_ref[...]
        agg = pa_ref[0] + pa_ref[1] + pb_ref[0] + pb_ref[1]
        nl2 = nl + jnp.maximum(
            _dot(nl, Wu1_ref[...]) + _dot(agg, Wu2_ref[...])
            + bu_ref[...][None, :], 0.0)
        out_ref[...] = _dot(nl2, Wd_ref[...]) + bd_ref[...][None, :]

    return pl.pallas_call(
        body,
        grid=(_N // _BN,),
        in_specs=[
            pl.BlockSpec((_BN, _LAT), lambda i: (i, 0)),
            _p_spec(), _p_spec(),
            _w_spec(), _w_spec(), _b_spec(), _w_spec(), _b_spec(),
        ],
        out_specs=pl.BlockSpec((_BN, _LAT), lambda i: (i, 0)),
        out_shape=jax.ShapeDtypeStruct((_N, _LAT), _F32),
    )(node_lat, pa, pb, Wu1, Wu2, b_upd, Wd_pad, bd_pad)


def kernel(position_sequence, particle_types, edge_index, emb_table,
           W_node_enc, b_node_enc, W_edge_enc, b_edge_enc,
           W_msg, b_msg, W_upd, b_upd, W_dec, b_dec):
    pos_flat = position_sequence.reshape(_N, 12)
    types_b = jnp.broadcast_to(
        particle_types.astype(jnp.int32)[:, None], (_N, _LAT))
    snd = edge_index[0].astype(jnp.int32)
    rcv = edge_index[1].astype(jnp.int32)
    lastpos = jnp.pad(pos_flat[:, 10:12], ((0, 0), (0, 14)))
    Wv = W_node_enc[:10]
    We = W_node_enc[10:]
    Wm1 = W_msg[:_LAT]
    Wm2 = W_msg[_LAT:2 * _LAT]
    Wm3 = W_msg[2 * _LAT:]
    Wu1 = W_upd[:_LAT]
    Wu2 = W_upd[_LAT:]
    Wd_pad = jnp.pad(W_dec, ((0, 0), (0, _LAT - W_dec.shape[1])))
    bd_pad = jnp.pad(b_dec, (0, _LAT - b_dec.shape[0]))
    M3 = jnp.pad(W_edge_enc, ((0, _LAT - W_edge_enc.shape[0]), (0, 0)))

    nl0, nlS0, nlR0 = _tc_node_encode(
        pos_flat, types_b, emb_table, Wv, We, b_node_enc, Wm2, Wm3)

    ga = _sc_gather(nlS0, nlR0, snd, rcv, 0, _H0, lastpos)
    gb = _sc_gather(nlS0, nlR0, snd, rcv, _H0, _H1, lastpos)
    e1a = _tc_edge_step1(*ga, M3, b_edge_enc, Wm1, b_msg)
    e1b = _tc_edge_step1(*gb, M3, b_edge_enc, Wm1, b_msg)
    pa1 = _sc_segment_sum(e1a, rcv, 0, _H0)
    pb1 = _sc_segment_sum(e1b, rcv, _H0, _H1)
    nl1, nlS1, nlR1 = _tc_node_update(nl0, pa1, pb1, Wu1, Wu2, b_upd,
                                      Wm2, Wm3)

    gS2a, gR2a = _sc_gather(nlS1, nlR1, snd, rcv, 0, _H0)
    gS2b, gR2b = _sc_gather(nlS1, nlR1, snd, rcv, _H0, _H1)
    e2a = _tc_edge_step2(e1a, gS2a, gR2a, Wm1, b_msg)
    e2b = _tc_edge_step2(e1b, gS2b, gR2b, Wm1, b_msg)
    pa2 = _sc_segment_sum(e2a, rcv, 0, _H0)
    pb2 = _sc_segment_sum(e2b, rcv, _H0, _H1)
    out_pad = _tc_node_final(nl1, pa2, pb2, Wu1, Wu2, b_upd, Wd_pad, bd_pad)

    return out_pad[:, :W_dec.shape[1]]


# speedup vs baseline: 5.7438x; 1.0416x over previous
import functools

import jax
import jax.numpy as jnp
from jax import lax
from jax.experimental import pallas as pl
from jax.experimental.pallas import tpu as pltpu
from jax.experimental.pallas import tpu_sc as plsc

_N = 10000
_E = 160000
_LAT = 128
_NC, _NS = 2, 16
_NW = _NC * _NS
_CHUNK = 40
_H0 = 83200
_H1 = _E - _H0
_BN = 1000
_BE = 3200
_F32 = jnp.float32


def _sc_mesh():
    return plsc.VectorSubcoreMesh(core_axis_name="c", subcore_axis_name="s",
                                  num_cores=_NC, num_subcores=_NS)


def _sc_gather(nlS, nlR, snd, rcv, start, count, lastpos=None):
    with_pos = lastpos is not None
    per_tile = count // _NW
    nchunk = per_tile // _CHUNK
    assert per_tile % _CHUNK == 0 and per_tile % 8 == 0
    out_type = [jax.ShapeDtypeStruct((count, _LAT), _F32),
                jax.ShapeDtypeStruct((count, _LAT), _F32)]
    scratch = [pltpu.VMEM((per_tile,), jnp.int32),
               pltpu.VMEM((per_tile,), jnp.int32)]
    nbuf = 4
    per_buf = [pltpu.VMEM((_CHUNK, _LAT), _F32),
               pltpu.VMEM((_CHUNK, _LAT), _F32)]
    if with_pos:
        out_type += [jax.ShapeDtypeStruct((count, _LAT), _F32)]
        per_buf += [pltpu.VMEM((_CHUNK, 16), _F32),
                    pltpu.VMEM((_CHUNK, 16), _F32)]
    nstream = len(per_buf)
    scratch += per_buf * nbuf
    scratch += [pltpu.SemaphoreType.DMA] * (2 * nbuf)

    @functools.partial(pl.kernel, out_type=out_type, mesh=_sc_mesh(),
                       scratch_types=scratch,
                       compiler_params=pltpu.CompilerParams(
                           use_tc_tiling_on_sc=False))
    def k(*refs):
        if with_pos:
            (nlS_h, nlR_h, snd_h, rcv_h, lp_h, gS_h, gR_h, rel_h,
             idxS, idxR, *rest) = refs
            tabs = (nlS_h, nlR_h, lp_h, lp_h)
            outs = (gS_h, gR_h, rel_h)
        else:
            (nlS_h, nlR_h, snd_h, rcv_h, gS_h, gR_h, idxS, idxR,
             *rest) = refs
            tabs = (nlS_h, nlR_h)
            outs = (gS_h, gR_h)
        bufs = [rest[b * nstream:(b + 1) * nstream] for b in range(nbuf)]
        sems = rest[nbuf * nstream:]
        sem_g = sems[:nbuf]
        sem_w = sems[nbuf:]
        idxs = (idxS, idxR, idxS, idxR)

        wid = lax.axis_index("s") * _NC + lax.axis_index("c")
        tile0 = wid * per_tile
        pltpu.sync_copy(snd_h.at[pl.ds(start + tile0, per_tile)], idxS)
        pltpu.sync_copy(rcv_h.at[pl.ds(start + tile0, per_tile)], idxR)

        def g_descs(c, b):
            ds_i = pl.ds(c * _CHUNK, _CHUNK)
            return [pltpu.make_async_copy(tabs[j].at[idxs[j].at[ds_i]],
                                          bufs[b][j], sem_g[b])
                    for j in range(nstream)]

        def w_descs(c, b):
            out_sl = pl.ds(tile0 + c * _CHUNK, _CHUNK)
            ds = []
            for j in range(len(outs)):
                dst = (outs[j].at[out_sl] if j < 2
                       else outs[j].at[out_sl, pl.ds(0, 16)])
                ds.append(pltpu.make_async_copy(bufs[b][j], dst, sem_w[b]))
            return ds

        def fire_g(c, b):
            for d in g_descs(c, b):
                d.start()

        for b in range(nbuf):
            fire_g(b, b)

        def step(c, b):
            for d in g_descs(c, b):
                d.wait()
            if with_pos:
                ps, pr = bufs[b][2], bufs[b][3]

                @pl.loop(0, _CHUNK)
                def _(r):
                    ps[r] = ps[r] - pr[r]

            for d in w_descs(c, b):
                d.start()
            for d in w_descs(c, b):
                d.wait()

            @pl.when(c + nbuf < nchunk)
            def _():
                fire_g(c + nbuf, b)

        @pl.loop(0, nchunk // nbuf)
        def _(cc):
            for b in range(nbuf):
                step(cc * nbuf + b, b)

        for c in range((nchunk // nbuf) * nbuf, nchunk):
            step(c, c % nbuf)

    if with_pos:
        return k(nlS, nlR, snd, rcv, lastpos)
    return k(nlS, nlR, snd, rcv)


def _sc_segment_sum(e_lat, rcv, start, count):
    nring = 4
    chunk = 40
    per_tile = count // _NW
    nchunk = per_tile // chunk
    assert per_tile % chunk == 0 and per_tile % 8 == 0

    @functools.partial(
        pl.kernel,
        out_type=jax.ShapeDtypeStruct((_NC, _N, _LAT), _F32),
        mesh=_sc_mesh(),
        scratch_types=[pltpu.VMEM_SHARED((_N, _LAT), _F32)]
        + [pltpu.VMEM((chunk,), jnp.int32)] * nring
        + [pltpu.VMEM((chunk, _LAT), _F32)] * nring
        + [pltpu.SemaphoreType.DMA] * (2 * nring))
    def k(e_h, rcv_h, out_h, acc_sh, *rest):
        idxs = rest[:nring]
        rows = rest[nring:2 * nring]
        semL = rest[2 * nring:3 * nring]
        semA = rest[3 * nring:]
        cid = lax.axis_index("c")
        sid = lax.axis_index("s")
        zb = rows[0]

        @pl.loop(0, chunk)
        def _(r):
            @pl.loop(0, _LAT // 16)
            def _(l):
                zb[r, pl.ds(l * 16, 16)] = jnp.zeros((16,), _F32)

        @pl.loop(0, 624 // chunk)
        def _(kk):
            pltpu.sync_copy(zb, acc_sh.at[pl.ds(sid * 624 + kk * chunk,
                                                chunk)])
        pltpu.sync_copy(zb.at[pl.ds(0, 624 % chunk)],
                        acc_sh.at[pl.ds(sid * 624 + (624 // chunk) * chunk,
                                        624 % chunk)])

        @pl.when(sid == _NS - 1)
        def _():
            pltpu.sync_copy(zb.at[pl.ds(0, _N - _NS * 624)],
                            acc_sh.at[pl.ds(_NS * 624, _N - _NS * 624)])

        plsc.subcore_barrier()
        base0 = cid * (count // _NC) + sid * per_tile

        def load_descs(c, b):
            base = base0 + c * chunk
            return [pltpu.make_async_copy(rcv_h.at[pl.ds(start + base,
                                                         chunk)],
                                          idxs[b], semL[b]),
                    pltpu.make_async_copy(e_h.at[pl.ds(base, chunk)],
                                          rows[b], semL[b])]

        def add_desc(b):
            return pltpu.make_async_copy(rows[b], acc_sh.at[idxs[b]],
                                         semA[b])

        def fire_loads(c, b):
            for d in load_descs(c, b):
                d.start()

        for b in range(nring):
            fire_loads(b, b)

        def step(c, b):
            for d in load_descs(c, b):
                d.wait()
            add_desc(b).start(add=True)
            add_desc(b).wait()

            @pl.when(c + nring < nchunk)
            def _():
                fire_loads(c + nring, b)

        @pl.loop(0, nchunk // nring)
        def _(cc):
            for b in range(nring):
                step(cc * nring + b, b)

        for c in range((nchunk // nring) * nring, nchunk):
            step(c, c % nring)

        plsc.subcore_barrier()
        pltpu.sync_copy(acc_sh.at[pl.ds(sid * 624, 624)],
                        out_h.at[cid, pl.ds(sid * 624, 624)])

        @pl.when(sid == _NS - 1)
        def _():
            pltpu.sync_copy(acc_sh.at[pl.ds(_NS * 624, _N - _NS * 624)],
                            out_h.at[cid, pl.ds(_NS * 624, _N - _NS * 624)])

    return k(e_lat, rcv)


def _dot(a, b):
    return jnp.dot(a, b, preferred_element_type=_F32)


def _w_spec():
    return pl.BlockSpec((_LAT, _LAT), lambda i: (0, 0))


def _b_spec():
    return pl.BlockSpec((_LAT,), lambda i: (0,))


def _e_spec():
    return pl.BlockSpec((_BE, _LAT), lambda i: (i, 0))


def _tc_node_encode(pos_flat, types_b, emb_table, Wv, We, b_enc, Wm2, Wm3):
    def body(pos_ref, t_ref, emb_ref, Wv_ref, We_ref, b_ref, W2_ref, W3_ref,
             nl_ref, s_ref, r_ref):
        pos = pos_ref[...]
        vel = pos[:, 2:12] - pos[:, 0:10]
        emb_proj = _dot(emb_ref[...], We_ref[...])
        t = t_ref[...]
        pre = _dot(vel, Wv_ref[...]) + b_ref[...][None, :]
        for kk in range(9):
            pre = pre + jnp.where(t == kk, emb_proj[kk][None, :], 0.0)
        nl = jnp.maximum(pre, 0.0)
        nl_ref[...] = nl
        s_ref[...] = _dot(nl, W2_ref[...])
        r_ref[...] = _dot(nl, W3_ref[...])

    return pl.pallas_call(
        body,
        grid=(_N // _BN,),
        in_specs=[
            pl.BlockSpec((_BN, 12), lambda i: (i, 0)),
            pl.BlockSpec((_BN, _LAT), lambda i: (i, 0)),
            pl.BlockSpec((9, 16), lambda i: (0, 0)),
            pl.BlockSpec((10, _LAT), lambda i: (0, 0)),
            pl.BlockSpec((16, _LAT), lambda i: (0, 0)),
            _b_spec(),
            _w_spec(),
            _w_spec(),
        ],
        out_specs=[pl.BlockSpec((_BN, _LAT), lambda i: (i, 0))] * 3,
        out_shape=[jax.ShapeDtypeStruct((_N, _LAT), _F32)] * 3,
    )(pos_flat, types_b, emb_table, Wv, We, b_enc, Wm2, Wm3)


def _tc_edge_step1(gS, gR, rel, M3, b_edge_enc, Wm1, b_msg):
    count = gS.shape[0]

    def body(gS_ref, gR_ref, rel_ref, M3_ref, be_ref, W1_ref, bm_ref,
             out_ref):
        lane = lax.broadcasted_iota(jnp.int32, (_BE, _LAT), 1)
        X = jnp.where(lane < 2, rel_ref[...], 0.0)
        d = jnp.sqrt(X[:, 0:1] * X[:, 0:1] + X[:, 1:2] * X[:, 1:2])
        XD = jnp.where(lane == 2, d, X)
        e0 = jnp.maximum(_dot(XD, M3_ref[...]) + be_ref[...][None, :], 0.0)
        pre = _dot(e0, W1_ref[...]) + bm_ref[...][None, :] \
            + gS_ref[...] + gR_ref[...]
        out_ref[...] = e0 + jnp.maximum(pre, 0.0)

    return pl.pallas_call(
        body,
        grid=(count // _BE,),
        in_specs=[
            _e_spec(),
            _e_spec(),
            _e_spec(),
            _w_spec(),
            _b_spec(),
            _w_spec(),
            _b_spec(),
        ],
        out_specs=_e_spec(),
        out_shape=jax.ShapeDtypeStruct((count, _LAT), _F32),
    )(gS, gR, rel, M3, b_edge_enc, Wm1, b_msg)


def _tc_edge_step2(e_prev, gS, gR, Wm1, b_msg):
    count = gS.shape[0]

    def body(e_ref, gS_ref, gR_ref, W1_ref, bm_ref, out_ref):
        e = e_ref[...]
        pre = _dot(e, W1_ref[...]) + bm_ref[...][None, :] \
            + gS_ref[...] + gR_ref[...]
        out_ref[...] = e + jnp.maximum(pre, 0.0)

    return pl.pallas_call(
        body,
        grid=(count // _BE,),
        in_specs=[_e_spec(), _e_spec(), _e_spec(), _w_spec(), _b_spec()],
        out_specs=_e_spec(),
        out_shape=jax.ShapeDtypeStruct((count, _LAT), _F32),
    )(e_prev, gS, gR, Wm1, b_msg)


def _p_spec():
    return pl.BlockSpec((_NC, _BN, _LAT), lambda i: (0, i, 0))


def _tc_node_update(node_lat, pa, pb, Wu1, Wu2, b_upd, Wm2, Wm3):
    def body(nl_ref, pa_ref, pb_ref, Wu1_ref, Wu2_ref, bu_ref, W2_ref,
             W3_ref, nl2_ref, s_ref, r_ref):
        nl = nl_ref[...]
        agg = pa_ref[0] + pa_ref[1] + pb_ref[0] + pb_ref[1]
        nl2 = nl + jnp.maximum(
            _dot(nl, Wu1_ref[...]) + _dot(agg, Wu2_ref[...])
            + bu_ref[...][None, :], 0.0)
        nl2_ref[...] = nl2
        s_ref[...] = _dot(nl2, W2_ref[...])
        r_ref[...] = _dot(nl2, W3_ref[...])

    return pl.pallas_call(
        body,
        grid=(_N // _BN,),
        in_specs=[
            pl.BlockSpec((_BN, _LAT), lambda i: (i, 0)),
            _p_spec(), _p_spec(),
            _w_spec(), _w_spec(), _b_spec(), _w_spec(), _w_spec(),
        ],
        out_specs=[pl.BlockSpec((_BN, _LAT), lambda i: (i, 0))] * 3,
        out_shape=[jax.ShapeDtypeStruct((_N, _LAT), _F32)] * 3,
    )(node_lat, pa, pb, Wu1, Wu2, b_upd, Wm2, Wm3)


def _tc_node_final(node_lat, pa, pb, Wu1, Wu2, b_upd, Wd_pad, bd_pad):
    def body(nl_ref, pa_ref, pb_ref, Wu1_ref, Wu2_ref, bu_ref, Wd_ref,
             bd_ref, out_ref):
        nl = nl_ref[...]
        agg = pa_ref[0] + pa_ref[1] + pb_ref[0] + pb_ref[1]
        nl2 = nl + jnp.maximum(
            _dot(nl, Wu1_ref[...]) + _dot(agg, Wu2_ref[...])
            + bu_ref[...][None, :], 0.0)
        out_ref[...] = _dot(nl2, Wd_ref[...]) + bd_ref[...][None, :]

    return pl.pallas_call(
        body,
        grid=(_N // _BN,),
        in_specs=[
            pl.BlockSpec((_BN, _LAT), lambda i: (i, 0)),
            _p_spec(), _p_spec(),
            _w_spec(), _w_spec(), _b_spec(), _w_spec(), _b_spec(),
        ],
        out_specs=pl.BlockSpec((_BN, _LAT), lambda i: (i, 0)),
        out_shape=jax.ShapeDtypeStruct((_N, _LAT), _F32),
    )(node_lat, pa, pb, Wu1, Wu2, b_upd, Wd_pad, bd_pad)


def kernel(position_sequence, particle_types, edge_index, emb_table,
           W_node_enc, b_node_enc, W_edge_enc, b_edge_enc,
           W_msg, b_msg, W_upd, b_upd, W_dec, b_dec):
    pos_flat = position_sequence.reshape(_N, 12)
    types_b = jnp.broadcast_to(
        particle_types.astype(jnp.int32)[:, None], (_N, _LAT))
    snd = edge_index[0].astype(jnp.int32)
    rcv = edge_index[1].astype(jnp.int32)
    lastpos = jnp.pad(pos_flat[:, 10:12], ((0, 0), (0, 14)))
    Wv = W_node_enc[:10]
    We = W_node_enc[10:]
    Wm1 = W_msg[:_LAT]
    Wm2 = W_msg[_LAT:2 * _LAT]
    Wm3 = W_msg[2 * _LAT:]
    Wu1 = W_upd[:_LAT]
    Wu2 = W_upd[_LAT:]
    Wd_pad = jnp.pad(W_dec, ((0, 0), (0, _LAT - W_dec.shape[1])))
    bd_pad = jnp.pad(b_dec, (0, _LAT - b_dec.shape[0]))
    M3 = jnp.pad(W_edge_enc, ((0, _LAT - W_edge_enc.shape[0]), (0, 0)))

    nl0, nlS0, nlR0 = _tc_node_encode(
        pos_flat, types_b, emb_table, Wv, We, b_node_enc, Wm2, Wm3)

    ga = _sc_gather(nlS0, nlR0, snd, rcv, 0, _H0, lastpos)
    gb = _sc_gather(nlS0, nlR0, snd, rcv, _H0, _H1, lastpos)
    e1a = _tc_edge_step1(*ga, M3, b_edge_enc, Wm1, b_msg)
    e1b = _tc_edge_step1(*gb, M3, b_edge_enc, Wm1, b_msg)
    pa1 = _sc_segment_sum(e1a, rcv, 0, _H0)
    pb1 = _sc_segment_sum(e1b, rcv, _H0, _H1)
    nl1, nlS1, nlR1 = _tc_node_update(nl0, pa1, pb1, Wu1, Wu2, b_upd,
                                      Wm2, Wm3)

    gS2a, gR2a = _sc_gather(nlS1, nlR1, snd, rcv, 0, _H0)
    gS2b, gR2b = _sc_gather(nlS1, nlR1, snd, rcv, _H0, _H1)
    e2a = _tc_edge_step2(e1a, gS2a, gR2a, Wm1, b_msg)
    e2b = _tc_edge_step2(e1b, gS2b, gR2b, Wm1, b_msg)
    pa2 = _sc_segment_sum(e2a, rcv, 0, _H0)
    pb2 = _sc_segment_sum(e2b, rcv, _H0, _H1)
    out_pad = _tc_node_final(nl1, pa2, pb2, Wu1, Wu2, b_upd, Wd_pad, bd_pad)

    return out_pad[:, :W_dec.shape[1]]
